# Initial kernel scaffold; baseline (speedup 1.0000x reference)
import jax
import jax.numpy as jnp
from jax.experimental import pallas as pl


def kernel(z, pos, edge_index, batch_ids, params):
    raise NotImplementedError("write your pallas kernel here")


# speedup vs baseline: 2.5306x; 2.5306x over previous
import functools

import jax
import jax.numpy as jnp
from jax import lax
from jax.experimental import pallas as pl
from jax.experimental.pallas import tpu as pltpu
from jax.experimental.pallas import tpu_sc as plsc

N_NODES = 50000
N_EDGES = 1600000
N_GRAPHS = 128
POS_DIM = 3
EMB_DIM = 24
M_DIM = 64
MLP_DIM = 256
N_OUT = 4
FOURIER = 4
EDGE_IN = FOURIER * 2 + 1 + EMB_DIM * 2

TROW = 32
MROW = 96
ACC_W = 24
PASS_W = 2 * ACC_W

NC, NS = 2, 16
NW = NC * NS

G_CHUNK = 2000
S_CHUNK = 2000
EB = 2000
NB = 5000


def _silu(x):
    return x * jax.nn.sigmoid(x)


def _gather_body(t_hbm, src_hbm, dst_hbm, a_out, b_out, idx_v, rows_v, sem):
    wid = lax.axis_index("s") * NC + lax.axis_index("c")
    n_iters = N_EDGES // (NW * G_CHUNK)
    base_w = wid * (N_EDGES // NW)

    def body(i, carry):
        base = base_w + i * G_CHUNK
        pltpu.sync_copy(src_hbm.at[pl.ds(base, G_CHUNK)], idx_v)
        pltpu.async_copy(t_hbm.at[idx_v], rows_v, sem).wait()
        pltpu.sync_copy(rows_v, a_out.at[pl.ds(base, G_CHUNK)])
        pltpu.sync_copy(dst_hbm.at[pl.ds(base, G_CHUNK)], idx_v)
        pltpu.async_copy(t_hbm.at[idx_v], rows_v, sem).wait()
        pltpu.sync_copy(rows_v, b_out.at[pl.ds(base, G_CHUNK)])
        return carry

    lax.fori_loop(0, n_iters, body, 0)


def _sc_mesh():
    return plsc.VectorSubcoreMesh(
        core_axis_name="c", subcore_axis_name="s",
        num_cores=NC, num_subcores=NS)


def _sc_gather(t, src, dst):
    fn = functools.partial(
        pl.kernel,
        out_type=(
            jax.ShapeDtypeStruct((N_EDGES, TROW), jnp.float32),
            jax.ShapeDtypeStruct((N_EDGES, TROW), jnp.float32),
        ),
        mesh=_sc_mesh(),
        compiler_params=pltpu.CompilerParams(use_tc_tiling_on_sc=False),
        scratch_types=[
            pltpu.VMEM((G_CHUNK,), jnp.int32),
            pltpu.VMEM((G_CHUNK, TROW), jnp.float32),
            pltpu.SemaphoreType.DMA,
        ],
    )(_gather_body)
    return fn(t, src, dst)


def _scatter_body(pass_base, msg_hbm, dst_hbm, zeros_hbm, acc_out,
                  idx_v, msg_v, acc_sh):
    c = lax.axis_index("c")
    s = lax.axis_index("s")
    col0 = pass_base + c * ACC_W

    @pl.when(s == 0)
    def _():
        pltpu.sync_copy(zeros_hbm, acc_sh)

    plsc.subcore_barrier()

    n_iters = N_EDGES // (NS * S_CHUNK)
    base_w = s * (N_EDGES // NS)

    def body(i, carry):
        base = base_w + i * S_CHUNK
        pltpu.sync_copy(dst_hbm.at[pl.ds(base, S_CHUNK)], idx_v)
        pltpu.sync_copy(
            msg_hbm.at[pl.ds(base, S_CHUNK), pl.ds(col0, ACC_W)], msg_v)
        pltpu.sync_copy(msg_v, acc_sh.at[idx_v], add=True)
        return carry

    lax.fori_loop(0, n_iters, body, 0)
    plsc.subcore_barrier()

    @pl.when(s == 0)
    def _():
        pltpu.sync_copy(acc_sh, acc_out.at[:, pl.ds(c * ACC_W, ACC_W)])


def _sc_scatter(msg, dst, zeros_acc, pass_idx):
    fn = functools.partial(
        pl.kernel,
        out_type=jax.ShapeDtypeStruct((N_NODES, PASS_W), jnp.float32),
        mesh=_sc_mesh(),
        compiler_params=pltpu.CompilerParams(use_tc_tiling_on_sc=False),
        scratch_types=[
            pltpu.VMEM((S_CHUNK,), jnp.int32),
            pltpu.VMEM((S_CHUNK, ACC_W), jnp.float32),
            pltpu.VMEM_SHARED((N_NODES, ACC_W), jnp.float32),
        ],
    )(functools.partial(_scatter_body, pass_idx * PASS_W))
    return fn(msg, dst, zeros_acc)


def _embed_tc(z_ref, pos_ref, wf_ref, bf_ref, t_ref):
    z = z_ref[...]
    onehot = (lax.broadcasted_iota(jnp.int32, (NB, 22), 1) == z
              ).astype(jnp.float32)
    feats = onehot @ wf_ref[...] + bf_ref[...]
    t_ref[:, 0:EMB_DIM] = feats
    t_ref[:, EMB_DIM:EMB_DIM + POS_DIM] = pos_ref[...]
    t_ref[:, EMB_DIM + POS_DIM:TROW] = jnp.zeros(
        (NB, TROW - EMB_DIM - POS_DIM), jnp.float32)


def _edge_tc(a_ref, b_ref, wd_ref, ws_ref, wsin_ref, wcos_ref, wd2_ref,
             eb1_ref, ew2_ref, eb2_ref, cw1_ref, cb1_ref, cw2_ref, cb2_ref,
             out_ref):
    a = a_ref[...]
    b = b_ref[...]
    fs = a[:, 0:EMB_DIM]
    fd = b[:, 0:EMB_DIM]
    pos_s = a[:, EMB_DIM:EMB_DIM + POS_DIM]
    pos_d = b[:, EMB_DIM:EMB_DIM + POS_DIM]
    rel = pos_s - pos_d
    d2 = jnp.sum(rel * rel, axis=1, keepdims=True)
    inv_scales = jnp.exp2(
        -lax.broadcasted_iota(jnp.int32, (1, FOURIER), 1).astype(jnp.float32))
    xs = d2 * inv_scales
    h = (fd @ wd_ref[...] + fs @ ws_ref[...]
         + jnp.sin(xs) @ wsin_ref[...] + jnp.cos(xs) @ wcos_ref[...]
         + d2 * wd2_ref[...] + eb1_ref[...])
    h = _silu(h)
    m = _silu(h @ ew2_ref[...] + eb2_ref[...])
    c1 = _silu(m @ cw1_ref[...] + cb1_ref[...])
    cw = (c1 @ cw2_ref[...] + cb2_ref[...])[:, 0:1]
    out_ref[:, 0:M_DIM] = m
    out_ref[:, M_DIM:M_DIM + POS_DIM] = cw * rel
    out_ref[:, M_DIM + POS_DIM:MROW] = jnp.zeros(
        (EB, MROW - M_DIM - POS_DIM), jnp.float32)


def _node_tc(t_ref, acc0_ref, acc1_ref, nw1a_ref, nw1b0_ref, nw1b1_ref,
             nb1_ref, nw2_ref, nb2_ref, tout_ref):
    t = t_ref[...]
    acc0 = acc0_ref[...]
    acc1 = acc1_ref[...]
    feats = t[:, 0:EMB_DIM]
    coors = t[:, EMB_DIM:EMB_DIM + POS_DIM]
    mhat = acc1[:, M_DIM - PASS_W:M_DIM - PASS_W + POS_DIM]
    nh = _silu(feats @ nw1a_ref[...] + acc0 @ nw1b0_ref[...]
               + acc1[:, 0:M_DIM - PASS_W] @ nw1b1_ref[...] + nb1_ref[...])
    hid = feats + nh @ nw2_ref[...] + nb2_ref[...]
    tout_ref[:, 0:EMB_DIM] = hid
    tout_ref[:, EMB_DIM:EMB_DIM + POS_DIM] = coors + mhat
    tout_ref[:, EMB_DIM + POS_DIM:TROW] = jnp.zeros(
        (NB, TROW - EMB_DIM - POS_DIM), jnp.float32)


def _final_tc(t0_ref, t1_ref, t2_ref, bid_ref,
              w1a_ref, w1b_ref, w1c_ref, b1_ref, w2_ref, b2_ref,
              w3_ref, b3_ref, w4_ref, b4_ref,
              out_ref, sums_acc, cnts_acc):
    i = pl.program_id(0)
    s0 = _silu(t0_ref[:, 0:EMB_DIM])
    s1 = _silu(t1_ref[:, 0:EMB_DIM])
    s2 = _silu(t2_ref[:, 0:EMB_DIM])
    h = _silu(s0 @ w1a_ref[...] + s1 @ w1b_ref[...] + s2 @ w1c_ref[...]
              + b1_ref[...])
    h = _silu(h @ w2_ref[...] + b2_ref[...])
    h = _silu(h @ w3_ref[...] + b3_ref[...])
    bid = bid_ref[...]
    onehot = (lax.broadcasted_iota(jnp.int32, (NB, N_GRAPHS), 1) == bid
              ).astype(jnp.float32)
    psums = lax.dot_general(onehot, h, (((0,), (0,)), ((), ())))
    pcnts = jnp.sum(onehot, axis=0, keepdims=True)

    @pl.when(i == 0)
    def _():
        sums_acc[...] = jnp.zeros_like(sums_acc)
        cnts_acc[...] = jnp.zeros_like(cnts_acc)

    sums_acc[...] += psums
    cnts_acc[...] += pcnts

    @pl.when(i == pl.num_programs(0) - 1)
    def _():
        cnts = jnp.maximum(cnts_acc[...], 1.0)
        mean = sums_acc[...] / cnts.reshape(N_GRAPHS, 1)
        out_ref[...] = jax.nn.sigmoid(mean @ w4_ref[...] + b4_ref[...])


def _full_spec(shape):
    return pl.BlockSpec(shape, lambda i: tuple(0 for _ in shape))


def kernel(z, pos, edge_index, batch_ids, params):
    f32 = jnp.float32
    src = edge_index[0]
    dst = edge_index[1]
    z2d = z.reshape(N_NODES, 1)
    bid2d = batch_ids.reshape(N_NODES, 1)

    wf = params['emb_table'] @ params['emb_W']
    bf = params['emb_b'].reshape(1, EMB_DIM)

    ks = []
    for p in params['kernels']:
        ew1 = p['eW1']
        h_pad = 128
        wd = jnp.zeros((EMB_DIM, h_pad), f32).at[:, :114].set(ew1[0:24])
        ws = jnp.zeros((EMB_DIM, h_pad), f32).at[:, :114].set(ew1[24:48])
        wsin = jnp.zeros((FOURIER, h_pad), f32).at[:, :114].set(ew1[48:52])
        wcos = jnp.zeros((FOURIER, h_pad), f32).at[:, :114].set(ew1[52:56])
        wd2 = jnp.zeros((1, h_pad), f32).at[:, :114].set(ew1[56:57])
        eb1 = jnp.zeros((1, h_pad), f32).at[:, :114].set(p['eb1'][None, :])
        ew2 = jnp.zeros((h_pad, M_DIM), f32).at[:114, :].set(p['eW2'])
        eb2 = p['eb2'].reshape(1, M_DIM)
        cw1 = p['cW1']
        cb1 = p['cb1'].reshape(1, M_DIM * 4)
        cw2 = jnp.zeros((M_DIM * 4, 8), f32).at[:, 0:1].set(p['cW2'])
        cb2 = jnp.zeros((1, 8), f32).at[:, 0:1].set(p['cb2'].reshape(1, 1))
        nw1a = p['nW1'][0:EMB_DIM]
        nw1b0 = p['nW1'][EMB_DIM:EMB_DIM + PASS_W]
        nw1b1 = p['nW1'][EMB_DIM + PASS_W:]
        nb1 = p['nb1'].reshape(1, EMB_DIM * 2)
        nw2 = p['nW2']
        nb2 = p['nb2'].reshape(1, EMB_DIM)
        ks.append((wd, ws, wsin, wcos, wd2, eb1, ew2, eb2, cw1, cb1, cw2,
                   cb2, nw1a, nw1b0, nw1b1, nb1, nw2, nb2))

    (fw1, fb1), (fw2, fb2), (fw3, fb3), (fw4, fb4) = params['ffnn']
    w1a, w1b, w1c = fw1[0:24], fw1[24:48], fw1[48:72]
    fb1 = fb1.reshape(1, MLP_DIM)
    fb2 = fb2.reshape(1, MLP_DIM)
    fb3 = fb3.reshape(1, MLP_DIM)
    fb4 = fb4.reshape(1, N_OUT)

    zeros_acc = jnp.zeros((N_NODES, ACC_W), jnp.float32)

    n_grid = N_NODES // NB
    t0 = pl.pallas_call(
        _embed_tc,
        grid=(n_grid,),
        in_specs=[
            pl.BlockSpec((NB, 1), lambda i: (i, 0)),
            pl.BlockSpec((NB, POS_DIM), lambda i: (i, 0)),
            _full_spec((22, EMB_DIM)),
            _full_spec((1, EMB_DIM)),
        ],
        out_specs=pl.BlockSpec((NB, TROW), lambda i: (i, 0)),
        out_shape=jax.ShapeDtypeStruct((N_NODES, TROW), f32),
    )(z2d, pos, wf, bf)

    tables = [t0]
    t_cur = t0
    for (wd, ws, wsin, wcos, wd2, eb1, ew2, eb2, cw1, cb1, cw2, cb2,
         nw1a, nw1b0, nw1b1, nb1, nw2, nb2) in ks:
        a_rows, b_rows = _sc_gather(t_cur, src, dst)

        e_grid = N_EDGES // EB
        msg = pl.pallas_call(
            _edge_tc,
            grid=(e_grid,),
            in_specs=[
                pl.BlockSpec((EB, TROW), lambda i: (i, 0)),
                pl.BlockSpec((EB, TROW), lambda i: (i, 0)),
                _full_spec((EMB_DIM, 128)), _full_spec((EMB_DIM, 128)),
                _full_spec((FOURIER, 128)), _full_spec((FOURIER, 128)),
                _full_spec((1, 128)), _full_spec((1, 128)),
                _full_spec((128, M_DIM)), _full_spec((1, M_DIM)),
                _full_spec((M_DIM, M_DIM * 4)), _full_spec((1, M_DIM * 4)),
                _full_spec((M_DIM * 4, 8)), _full_spec((1, 8)),
            ],
            out_specs=pl.BlockSpec((EB, MROW), lambda i: (i, 0)),
            out_shape=jax.ShapeDtypeStruct((N_EDGES, MROW), f32),
        )(a_rows, b_rows, wd, ws, wsin, wcos, wd2, eb1, ew2, eb2,
          cw1, cb1, cw2, cb2)

        acc0 = _sc_scatter(msg, dst, zeros_acc, 0)
        acc1 = _sc_scatter(msg, dst, zeros_acc, 1)

        t_cur = pl.pallas_call(
            _node_tc,
            grid=(n_grid,),
            in_specs=[
                pl.BlockSpec((NB, TROW), lambda i: (i, 0)),
                pl.BlockSpec((NB, PASS_W), lambda i: (i, 0)),
                pl.BlockSpec((NB, PASS_W), lambda i: (i, 0)),
                _full_spec((EMB_DIM, EMB_DIM * 2)),
                _full_spec((PASS_W, EMB_DIM * 2)),
                _full_spec((M_DIM - PASS_W, EMB_DIM * 2)),
                _full_spec((1, EMB_DIM * 2)),
                _full_spec((EMB_DIM * 2, EMB_DIM)),
                _full_spec((1, EMB_DIM)),
            ],
            out_specs=pl.BlockSpec((NB, TROW), lambda i: (i, 0)),
            out_shape=jax.ShapeDtypeStruct((N_NODES, TROW), f32),
        )(t_cur, acc0, acc1, nw1a, nw1b0, nw1b1, nb1, nw2, nb2)
        tables.append(t_cur)

    out = pl.pallas_call(
        _final_tc,
        grid=(n_grid,),
        in_specs=[
            pl.BlockSpec((NB, TROW), lambda i: (i, 0)),
            pl.BlockSpec((NB, TROW), lambda i: (i, 0)),
            pl.BlockSpec((NB, TROW), lambda i: (i, 0)),
            pl.BlockSpec((NB, 1), lambda i: (i, 0)),
            _full_spec((EMB_DIM, MLP_DIM)), _full_spec((EMB_DIM, MLP_DIM)),
            _full_spec((EMB_DIM, MLP_DIM)), _full_spec((1, MLP_DIM)),
            _full_spec((MLP_DIM, MLP_DIM)), _full_spec((1, MLP_DIM)),
            _full_spec((MLP_DIM, MLP_DIM)), _full_spec((1, MLP_DIM)),
            _full_spec((MLP_DIM, N_OUT)), _full_spec((1, N_OUT)),
        ],
        out_specs=pl.BlockSpec((N_GRAPHS, N_OUT), lambda i: (0, 0)),
        out_shape=jax.ShapeDtypeStruct((N_GRAPHS, N_OUT), f32),
        scratch_shapes=[
            pltpu.VMEM((N_GRAPHS, MLP_DIM), f32),
            pltpu.VMEM((1, N_GRAPHS), f32),
        ],
    )(tables[0], tables[1], tables[2], bid2d,
      w1a, w1b, w1c, fb1, fw2, fb2, fw3, fb3, fw4, fb4)
    return out


# speedup vs baseline: 2.5600x; 1.0116x over previous
import functools

import jax
import jax.numpy as jnp
from jax import lax
from jax.experimental import pallas as pl
from jax.experimental.pallas import tpu as pltpu
from jax.experimental.pallas import tpu_sc as plsc

N_NODES = 50000
N_EDGES = 1600000
N_GRAPHS = 128
POS_DIM = 3
EMB_DIM = 24
M_DIM = 64
MLP_DIM = 256
N_OUT = 4
FOURIER = 4
EDGE_IN = FOURIER * 2 + 1 + EMB_DIM * 2

TROW = 32
MROW = 96
ACC_W = 24
PASS_W = 2 * ACC_W

NC, NS = 2, 16
NW = NC * NS

G_CHUNK = 2000
S_CHUNK = 2000
EB = 2000
NB = 5000


def _sigmoid(x):
    return 0.5 * (jnp.tanh(x * 0.5) + 1.0)


def _silu(x):
    return x * _sigmoid(x)


def _gather_body(t_hbm, src_hbm, dst_hbm, a_out, b_out, idx_v, rows_v, sem):
    wid = lax.axis_index("s") * NC + lax.axis_index("c")
    n_iters = N_EDGES // (NW * G_CHUNK)
    base_w = wid * (N_EDGES // NW)

    def body(i, carry):
        base = base_w + i * G_CHUNK
        pltpu.sync_copy(src_hbm.at[pl.ds(base, G_CHUNK)], idx_v)
        pltpu.async_copy(t_hbm.at[idx_v], rows_v, sem).wait()
        pltpu.sync_copy(rows_v, a_out.at[pl.ds(base, G_CHUNK)])
        pltpu.sync_copy(dst_hbm.at[pl.ds(base, G_CHUNK)], idx_v)
        pltpu.async_copy(t_hbm.at[idx_v], rows_v, sem).wait()
        pltpu.sync_copy(rows_v, b_out.at[pl.ds(base, G_CHUNK)])
        return carry

    lax.fori_loop(0, n_iters, body, 0)


def _sc_mesh():
    return plsc.VectorSubcoreMesh(
        core_axis_name="c", subcore_axis_name="s",
        num_cores=NC, num_subcores=NS)


def _sc_gather(t, src, dst):
    fn = functools.partial(
        pl.kernel,
        out_type=(
            jax.ShapeDtypeStruct((N_EDGES, TROW), jnp.float32),
            jax.ShapeDtypeStruct((N_EDGES, TROW), jnp.float32),
        ),
        mesh=_sc_mesh(),
        compiler_params=pltpu.CompilerParams(use_tc_tiling_on_sc=False),
        scratch_types=[
            pltpu.VMEM((G_CHUNK,), jnp.int32),
            pltpu.VMEM((G_CHUNK, TROW), jnp.float32),
            pltpu.SemaphoreType.DMA,
        ],
    )(_gather_body)
    return fn(t, src, dst)


def _scatter_body(pass_base, msg_hbm, dst_hbm, zeros_hbm, acc_out,
                  idx_v, msg_v, acc_sh):
    c = lax.axis_index("c")
    s = lax.axis_index("s")
    col0 = pass_base + c * ACC_W

    @pl.when(s == 0)
    def _():
        pltpu.sync_copy(zeros_hbm, acc_sh)

    plsc.subcore_barrier()

    n_iters = N_EDGES // (NS * S_CHUNK)
    base_w = s * (N_EDGES // NS)

    def body(i, carry):
        base = base_w + i * S_CHUNK
        pltpu.sync_copy(dst_hbm.at[pl.ds(base, S_CHUNK)], idx_v)
        pltpu.sync_copy(
            msg_hbm.at[pl.ds(base, S_CHUNK), pl.ds(col0, ACC_W)], msg_v)
        pltpu.sync_copy(msg_v, acc_sh.at[idx_v], add=True)
        return carry

    lax.fori_loop(0, n_iters, body, 0)
    plsc.subcore_barrier()

    @pl.when(s == 0)
    def _():
        pltpu.sync_copy(acc_sh, acc_out.at[:, pl.ds(c * ACC_W, ACC_W)])


def _sc_scatter(msg, dst, zeros_acc, pass_idx):
    fn = functools.partial(
        pl.kernel,
        out_type=jax.ShapeDtypeStruct((N_NODES, PASS_W), jnp.float32),
        mesh=_sc_mesh(),
        compiler_params=pltpu.CompilerParams(use_tc_tiling_on_sc=False),
        scratch_types=[
            pltpu.VMEM((S_CHUNK,), jnp.int32),
            pltpu.VMEM((S_CHUNK, ACC_W), jnp.float32),
            pltpu.VMEM_SHARED((N_NODES, ACC_W), jnp.float32),
        ],
    )(functools.partial(_scatter_body, pass_idx * PASS_W))
    return fn(msg, dst, zeros_acc)


def _embed_tc(z_ref, pos_ref, wf_ref, bf_ref, t_ref):
    z = z_ref[...]
    onehot = (lax.broadcasted_iota(jnp.int32, (NB, 22), 1) == z
              ).astype(jnp.float32)
    feats = onehot @ wf_ref[...] + bf_ref[...]
    t_ref[:, 0:EMB_DIM] = feats
    t_ref[:, EMB_DIM:EMB_DIM + POS_DIM] = pos_ref[...]
    t_ref[:, EMB_DIM + POS_DIM:TROW] = jnp.zeros(
        (NB, TROW - EMB_DIM - POS_DIM), jnp.float32)


def _edge_tc(a_ref, b_ref, wd_ref, ws_ref, wsin_ref, wcos_ref, wd2_ref,
             eb1_ref, ew2_ref, eb2_ref, cw1_ref, cb1_ref, cw2_ref, cb2_ref,
             out_ref):
    a = a_ref[...]
    b = b_ref[...]
    fs = a[:, 0:EMB_DIM]
    fd = b[:, 0:EMB_DIM]
    pos_s = a[:, EMB_DIM:EMB_DIM + POS_DIM]
    pos_d = b[:, EMB_DIM:EMB_DIM + POS_DIM]
    rel = pos_s - pos_d
    d2 = jnp.sum(rel * rel, axis=1, keepdims=True)
    inv_scales = jnp.exp2(
        -lax.broadcasted_iota(jnp.int32, (1, FOURIER), 1).astype(jnp.float32))
    xs = d2 * inv_scales
    h = (fd @ wd_ref[...] + fs @ ws_ref[...]
         + jnp.sin(xs) @ wsin_ref[...] + jnp.cos(xs) @ wcos_ref[...]
         + d2 * wd2_ref[...] + eb1_ref[...])
    h = _silu(h)
    m = _silu(h @ ew2_ref[...] + eb2_ref[...])
    c1 = _silu(m @ cw1_ref[...] + cb1_ref[...])
    cw = (c1 @ cw2_ref[...] + cb2_ref[...])[:, 0:1]
    out_ref[:, 0:M_DIM] = m
    out_ref[:, M_DIM:M_DIM + POS_DIM] = cw * rel
    out_ref[:, M_DIM + POS_DIM:MROW] = jnp.zeros(
        (EB, MROW - M_DIM - POS_DIM), jnp.float32)


def _node_tc(t_ref, acc0_ref, acc1_ref, nw1a_ref, nw1b0_ref, nw1b1_ref,
             nb1_ref, nw2_ref, nb2_ref, tout_ref):
    t = t_ref[...]
    acc0 = acc0_ref[...]
    acc1 = acc1_ref[...]
    feats = t[:, 0:EMB_DIM]
    coors = t[:, EMB_DIM:EMB_DIM + POS_DIM]
    mhat = acc1[:, M_DIM - PASS_W:M_DIM - PASS_W + POS_DIM]
    nh = _silu(feats @ nw1a_ref[...] + acc0 @ nw1b0_ref[...]
               + acc1[:, 0:M_DIM - PASS_W] @ nw1b1_ref[...] + nb1_ref[...])
    hid = feats + nh @ nw2_ref[...] + nb2_ref[...]
    tout_ref[:, 0:EMB_DIM] = hid
    tout_ref[:, EMB_DIM:EMB_DIM + POS_DIM] = coors + mhat
    tout_ref[:, EMB_DIM + POS_DIM:TROW] = jnp.zeros(
        (NB, TROW - EMB_DIM - POS_DIM), jnp.float32)


def _final_tc(t0_ref, t1_ref, t2_ref, bid_ref,
              w1a_ref, w1b_ref, w1c_ref, b1_ref, w2_ref, b2_ref,
              w3_ref, b3_ref, w4_ref, b4_ref,
              out_ref, sums_acc, cnts_acc):
    i = pl.program_id(0)
    s0 = _silu(t0_ref[:, 0:EMB_DIM])
    s1 = _silu(t1_ref[:, 0:EMB_DIM])
    s2 = _silu(t2_ref[:, 0:EMB_DIM])
    h = _silu(s0 @ w1a_ref[...] + s1 @ w1b_ref[...] + s2 @ w1c_ref[...]
              + b1_ref[...])
    h = _silu(h @ w2_ref[...] + b2_ref[...])
    h = _silu(h @ w3_ref[...] + b3_ref[...])
    bid = bid_ref[...]
    onehot = (lax.broadcasted_iota(jnp.int32, (NB, N_GRAPHS), 1) == bid
              ).astype(jnp.float32)
    psums = lax.dot_general(onehot, h, (((0,), (0,)), ((), ())))
    pcnts = jnp.sum(onehot, axis=0, keepdims=True)

    @pl.when(i == 0)
    def _():
        sums_acc[...] = jnp.zeros_like(sums_acc)
        cnts_acc[...] = jnp.zeros_like(cnts_acc)

    sums_acc[...] += psums
    cnts_acc[...] += pcnts

    @pl.when(i == pl.num_programs(0) - 1)
    def _():
        cnts = jnp.maximum(cnts_acc[...], 1.0)
        mean = sums_acc[...] / cnts.reshape(N_GRAPHS, 1)
        out_ref[...] = jax.nn.sigmoid(mean @ w4_ref[...] + b4_ref[...])


def _full_spec(shape):
    return pl.BlockSpec(shape, lambda i: tuple(0 for _ in shape))


def kernel(z, pos, edge_index, batch_ids, params):
    f32 = jnp.float32
    src = edge_index[0]
    dst = edge_index[1]
    z2d = z.reshape(N_NODES, 1)
    bid2d = batch_ids.reshape(N_NODES, 1)

    wf = params['emb_table'] @ params['emb_W']
    bf = params['emb_b'].reshape(1, EMB_DIM)

    ks = []
    for p in params['kernels']:
        ew1 = p['eW1']
        h_pad = 128
        wd = jnp.zeros((EMB_DIM, h_pad), f32).at[:, :114].set(ew1[0:24])
        ws = jnp.zeros((EMB_DIM, h_pad), f32).at[:, :114].set(ew1[24:48])
        wsin = jnp.zeros((FOURIER, h_pad), f32).at[:, :114].set(ew1[48:52])
        wcos = jnp.zeros((FOURIER, h_pad), f32).at[:, :114].set(ew1[52:56])
        wd2 = jnp.zeros((1, h_pad), f32).at[:, :114].set(ew1[56:57])
        eb1 = jnp.zeros((1, h_pad), f32).at[:, :114].set(p['eb1'][None, :])
        ew2 = jnp.zeros((h_pad, M_DIM), f32).at[:114, :].set(p['eW2'])
        eb2 = p['eb2'].reshape(1, M_DIM)
        cw1 = p['cW1']
        cb1 = p['cb1'].reshape(1, M_DIM * 4)
        cw2 = jnp.zeros((M_DIM * 4, 8), f32).at[:, 0:1].set(p['cW2'])
        cb2 = jnp.zeros((1, 8), f32).at[:, 0:1].set(p['cb2'].reshape(1, 1))
        nw1a = p['nW1'][0:EMB_DIM]
        nw1b0 = p['nW1'][EMB_DIM:EMB_DIM + PASS_W]
        nw1b1 = p['nW1'][EMB_DIM + PASS_W:]
        nb1 = p['nb1'].reshape(1, EMB_DIM * 2)
        nw2 = p['nW2']
        nb2 = p['nb2'].reshape(1, EMB_DIM)
        ks.append((wd, ws, wsin, wcos, wd2, eb1, ew2, eb2, cw1, cb1, cw2,
                   cb2, nw1a, nw1b0, nw1b1, nb1, nw2, nb2))

    (fw1, fb1), (fw2, fb2), (fw3, fb3), (fw4, fb4) = params['ffnn']
    w1a, w1b, w1c = fw1[0:24], fw1[24:48], fw1[48:72]
    fb1 = fb1.reshape(1, MLP_DIM)
    fb2 = fb2.reshape(1, MLP_DIM)
    fb3 = fb3.reshape(1, MLP_DIM)
    fb4 = fb4.reshape(1, N_OUT)

    zeros_acc = jnp.zeros((N_NODES, ACC_W), jnp.float32)

    n_grid = N_NODES // NB
    t0 = pl.pallas_call(
        _embed_tc,
        grid=(n_grid,),
        in_specs=[
            pl.BlockSpec((NB, 1), lambda i: (i, 0)),
            pl.BlockSpec((NB, POS_DIM), lambda i: (i, 0)),
            _full_spec((22, EMB_DIM)),
            _full_spec((1, EMB_DIM)),
        ],
        out_specs=pl.BlockSpec((NB, TROW), lambda i: (i, 0)),
        out_shape=jax.ShapeDtypeStruct((N_NODES, TROW), f32),
    )(z2d, pos, wf, bf)

    tables = [t0]
    t_cur = t0
    for (wd, ws, wsin, wcos, wd2, eb1, ew2, eb2, cw1, cb1, cw2, cb2,
         nw1a, nw1b0, nw1b1, nb1, nw2, nb2) in ks:
        a_rows, b_rows = _sc_gather(t_cur, src, dst)

        e_grid = N_EDGES // EB
        msg = pl.pallas_call(
            _edge_tc,
            grid=(e_grid,),
            in_specs=[
                pl.BlockSpec((EB, TROW), lambda i: (i, 0)),
                pl.BlockSpec((EB, TROW), lambda i: (i, 0)),
                _full_spec((EMB_DIM, 128)), _full_spec((EMB_DIM, 128)),
                _full_spec((FOURIER, 128)), _full_spec((FOURIER, 128)),
                _full_spec((1, 128)), _full_spec((1, 128)),
                _full_spec((128, M_DIM)), _full_spec((1, M_DIM)),
                _full_spec((M_DIM, M_DIM * 4)), _full_spec((1, M_DIM * 4)),
                _full_spec((M_DIM * 4, 8)), _full_spec((1, 8)),
            ],
            out_specs=pl.BlockSpec((EB, MROW), lambda i: (i, 0)),
            out_shape=jax.ShapeDtypeStruct((N_EDGES, MROW), f32),
        )(a_rows, b_rows, wd, ws, wsin, wcos, wd2, eb1, ew2, eb2,
          cw1, cb1, cw2, cb2)

        acc0 = _sc_scatter(msg, dst, zeros_acc, 0)
        acc1 = _sc_scatter(msg, dst, zeros_acc, 1)

        t_cur = pl.pallas_call(
            _node_tc,
            grid=(n_grid,),
            in_specs=[
                pl.BlockSpec((NB, TROW), lambda i: (i, 0)),
                pl.BlockSpec((NB, PASS_W), lambda i: (i, 0)),
                pl.BlockSpec((NB, PASS_W), lambda i: (i, 0)),
                _full_spec((EMB_DIM, EMB_DIM * 2)),
                _full_spec((PASS_W, EMB_DIM * 2)),
                _full_spec((M_DIM - PASS_W, EMB_DIM * 2)),
                _full_spec((1, EMB_DIM * 2)),
                _full_spec((EMB_DIM * 2, EMB_DIM)),
                _full_spec((1, EMB_DIM)),
            ],
            out_specs=pl.BlockSpec((NB, TROW), lambda i: (i, 0)),
            out_shape=jax.ShapeDtypeStruct((N_NODES, TROW), f32),
        )(t_cur, acc0, acc1, nw1a, nw1b0, nw1b1, nb1, nw2, nb2)
        tables.append(t_cur)

    out = pl.pallas_call(
        _final_tc,
        grid=(n_grid,),
        in_specs=[
            pl.BlockSpec((NB, TROW), lambda i: (i, 0)),
            pl.BlockSpec((NB, TROW), lambda i: (i, 0)),
            pl.BlockSpec((NB, TROW), lambda i: (i, 0)),
            pl.BlockSpec((NB, 1), lambda i: (i, 0)),
            _full_spec((EMB_DIM, MLP_DIM)), _full_spec((EMB_DIM, MLP_DIM)),
            _full_spec((EMB_DIM, MLP_DIM)), _full_spec((1, MLP_DIM)),
            _full_spec((MLP_DIM, MLP_DIM)), _full_spec((1, MLP_DIM)),
            _full_spec((MLP_DIM, MLP_DIM)), _full_spec((1, MLP_DIM)),
            _full_spec((MLP_DIM, N_OUT)), _full_spec((1, N_OUT)),
        ],
        out_specs=pl.BlockSpec((N_GRAPHS, N_OUT), lambda i: (0, 0)),
        out_shape=jax.ShapeDtypeStruct((N_GRAPHS, N_OUT), f32),
        scratch_shapes=[
            pltpu.VMEM((N_GRAPHS, MLP_DIM), f32),
            pltpu.VMEM((1, N_GRAPHS), f32),
        ],
    )(tables[0], tables[1], tables[2], bid2d,
      w1a, w1b, w1c, fb1, fw2, fb2, fw3, fb3, fw4, fb4)
    return out


# speedup vs baseline: 2.8511x; 1.1137x over previous
import functools

import jax
import jax.numpy as jnp
from jax import lax
from jax.experimental import pallas as pl
from jax.experimental.pallas import tpu as pltpu
from jax.experimental.pallas import tpu_sc as plsc

N_NODES = 50000
N_EDGES = 1600000
N_GRAPHS = 128
POS_DIM = 3
EMB_DIM = 24
M_DIM = 64
MLP_DIM = 256
N_OUT = 4
FOURIER = 4
EDGE_IN = FOURIER * 2 + 1 + EMB_DIM * 2

TROW = 32
MROW = 96
ACC_W = 24
PASS_W = 2 * ACC_W

NC, NS = 2, 16
NW = NC * NS

G_CHUNK = 2000
S_CHUNK = 2000
EB = 2000
NB = 5000


def _sigmoid(x):
    return 0.5 * (jnp.tanh(x * 0.5) + 1.0)


def _silu(x):
    return x * _sigmoid(x)


def _gather_body(t_hbm, src_hbm, dst_hbm, a_out, b_out, idx_v, rows_v, sem):
    wid = lax.axis_index("s") * NC + lax.axis_index("c")
    n_iters = N_EDGES // (NW * G_CHUNK)
    base_w = wid * (N_EDGES // NW)

    def body(i, carry):
        base = base_w + i * G_CHUNK
        pltpu.sync_copy(src_hbm.at[pl.ds(base, G_CHUNK)], idx_v)
        pltpu.async_copy(t_hbm.at[idx_v], rows_v, sem).wait()
        pltpu.sync_copy(rows_v, a_out.at[pl.ds(base, G_CHUNK)])
        pltpu.sync_copy(dst_hbm.at[pl.ds(base, G_CHUNK)], idx_v)
        pltpu.async_copy(t_hbm.at[idx_v], rows_v, sem).wait()
        pltpu.sync_copy(rows_v, b_out.at[pl.ds(base, G_CHUNK)])
        return carry

    lax.fori_loop(0, n_iters, body, 0)


def _sc_mesh():
    return plsc.VectorSubcoreMesh(
        core_axis_name="c", subcore_axis_name="s",
        num_cores=NC, num_subcores=NS)


def _sc_gather(t, src, dst):
    fn = functools.partial(
        pl.kernel,
        out_type=(
            jax.ShapeDtypeStruct((N_EDGES, TROW), jnp.float32),
            jax.ShapeDtypeStruct((N_EDGES, TROW), jnp.float32),
        ),
        mesh=_sc_mesh(),
        compiler_params=pltpu.CompilerParams(use_tc_tiling_on_sc=False),
        scratch_types=[
            pltpu.VMEM((G_CHUNK,), jnp.int32),
            pltpu.VMEM((G_CHUNK, TROW), jnp.float32),
            pltpu.SemaphoreType.DMA,
        ],
    )(_gather_body)
    return fn(t, src, dst)


def _scatter_body(pass_base, msg_hbm, dst_hbm, zeros_hbm, acc_out,
                  idx_v, msg_v, acc_sh):
    c = lax.axis_index("c")
    s = lax.axis_index("s")
    col0 = pass_base + c * ACC_W

    @pl.when(s == 0)
    def _():
        pltpu.sync_copy(zeros_hbm, acc_sh)

    plsc.subcore_barrier()

    n_iters = N_EDGES // (NS * S_CHUNK)
    base_w = s * (N_EDGES // NS)

    def body(i, carry):
        base = base_w + i * S_CHUNK
        pltpu.sync_copy(dst_hbm.at[pl.ds(base, S_CHUNK)], idx_v)
        pltpu.sync_copy(
            msg_hbm.at[pl.ds(base, S_CHUNK), pl.ds(col0, ACC_W)], msg_v)
        pltpu.sync_copy(msg_v, acc_sh.at[idx_v], add=True)
        return carry

    lax.fori_loop(0, n_iters, body, 0)
    plsc.subcore_barrier()

    @pl.when(s == 0)
    def _():
        pltpu.sync_copy(acc_sh, acc_out.at[:, pl.ds(c * ACC_W, ACC_W)])


def _sc_scatter(msg, dst, zeros_acc, pass_idx):
    fn = functools.partial(
        pl.kernel,
        out_type=jax.ShapeDtypeStruct((N_NODES, PASS_W), jnp.float32),
        mesh=_sc_mesh(),
        compiler_params=pltpu.CompilerParams(use_tc_tiling_on_sc=False),
        scratch_types=[
            pltpu.VMEM((S_CHUNK,), jnp.int32),
            pltpu.VMEM((S_CHUNK, ACC_W), jnp.float32),
            pltpu.VMEM_SHARED((N_NODES, ACC_W), jnp.float32),
        ],
    )(functools.partial(_scatter_body, pass_idx * PASS_W))
    return fn(msg, dst, zeros_acc)


def _embed_tc(z_ref, pos_ref, wf_ref, bf_ref, t_ref):
    z = z_ref[...]
    onehot = (lax.broadcasted_iota(jnp.int32, (NB, 22), 1) == z
              ).astype(jnp.float32)
    feats = onehot @ wf_ref[...] + bf_ref[...]
    t_ref[:, 0:EMB_DIM] = feats
    t_ref[:, EMB_DIM:EMB_DIM + POS_DIM] = pos_ref[...]
    t_ref[:, EMB_DIM + POS_DIM:TROW] = jnp.zeros(
        (NB, TROW - EMB_DIM - POS_DIM), jnp.float32)


def _edge_tc(a_ref, b_ref, wa_ref, wb_ref, wf_ref, op_ref, ph_ref,
             eb1_ref, ew2_ref, eb2_ref, cw1_ref, cb1_ref, cw2_ref, cb2_ref,
             out_ref):
    a = a_ref[...]
    b = b_ref[...]
    dd = a - b
    ang = (dd * dd) @ op_ref[...]
    lane = lax.broadcasted_iota(jnp.int32, (1, 16), 1)
    trig = jnp.where(lane == 2 * FOURIER, ang, jnp.sin(ang + ph_ref[...]))
    h = _silu(a @ wa_ref[...] + b @ wb_ref[...] + trig @ wf_ref[...]
              + eb1_ref[...])
    m = _silu(h @ ew2_ref[...] + eb2_ref[...])
    c1 = _silu(m @ cw1_ref[...] + cb1_ref[...])
    cw8 = c1 @ cw2_ref[...] + cb2_ref[...]
    out_ref[:, 0:M_DIM] = m
    out_ref[:, M_DIM:M_DIM + POS_DIM] = (
        cw8[:, 0:POS_DIM] * dd[:, EMB_DIM:EMB_DIM + POS_DIM])


def _node_tc(t_ref, acc0_ref, acc1_ref, nw1a_ref, nw1b0_ref, nw1b1_ref,
             nb1_ref, nw2_ref, nb2_ref, tout_ref):
    t = t_ref[...]
    acc0 = acc0_ref[...]
    acc1 = acc1_ref[...]
    feats = t[:, 0:EMB_DIM]
    coors = t[:, EMB_DIM:EMB_DIM + POS_DIM]
    mhat = acc1[:, M_DIM - PASS_W:M_DIM - PASS_W + POS_DIM]
    nh = _silu(feats @ nw1a_ref[...] + acc0 @ nw1b0_ref[...]
               + acc1[:, 0:M_DIM - PASS_W] @ nw1b1_ref[...] + nb1_ref[...])
    hid = feats + nh @ nw2_ref[...] + nb2_ref[...]
    tout_ref[:, 0:EMB_DIM] = hid
    tout_ref[:, EMB_DIM:EMB_DIM + POS_DIM] = coors + mhat
    tout_ref[:, EMB_DIM + POS_DIM:TROW] = jnp.zeros(
        (NB, TROW - EMB_DIM - POS_DIM), jnp.float32)


def _final_tc(t0_ref, t1_ref, t2_ref, bid_ref,
              w1a_ref, w1b_ref, w1c_ref, b1_ref, w2_ref, b2_ref,
              w3_ref, b3_ref, w4_ref, b4_ref,
              out_ref, sums_acc, cnts_acc):
    i = pl.program_id(0)
    s0 = _silu(t0_ref[:, 0:EMB_DIM])
    s1 = _silu(t1_ref[:, 0:EMB_DIM])
    s2 = _silu(t2_ref[:, 0:EMB_DIM])
    h = _silu(s0 @ w1a_ref[...] + s1 @ w1b_ref[...] + s2 @ w1c_ref[...]
              + b1_ref[...])
    h = _silu(h @ w2_ref[...] + b2_ref[...])
    h = _silu(h @ w3_ref[...] + b3_ref[...])
    bid = bid_ref[...]
    onehot = (lax.broadcasted_iota(jnp.int32, (NB, N_GRAPHS), 1) == bid
              ).astype(jnp.float32)
    psums = lax.dot_general(onehot, h, (((0,), (0,)), ((), ())))
    pcnts = jnp.sum(onehot, axis=0, keepdims=True)

    @pl.when(i == 0)
    def _():
        sums_acc[...] = jnp.zeros_like(sums_acc)
        cnts_acc[...] = jnp.zeros_like(cnts_acc)

    sums_acc[...] += psums
    cnts_acc[...] += pcnts

    @pl.when(i == pl.num_programs(0) - 1)
    def _():
        cnts = jnp.maximum(cnts_acc[...], 1.0)
        mean = sums_acc[...] / cnts.reshape(N_GRAPHS, 1)
        out_ref[...] = jax.nn.sigmoid(mean @ w4_ref[...] + b4_ref[...])


def _full_spec(shape):
    return pl.BlockSpec(shape, lambda i: tuple(0 for _ in shape))


def kernel(z, pos, edge_index, batch_ids, params):
    f32 = jnp.float32
    src = edge_index[0]
    dst = edge_index[1]
    z2d = z.reshape(N_NODES, 1)
    bid2d = batch_ids.reshape(N_NODES, 1)

    wf = params['emb_table'] @ params['emb_W']
    bf = params['emb_b'].reshape(1, EMB_DIM)

    ks = []
    for p in params['kernels']:
        ew1 = p['eW1']
        h_pad = 128
        wa = jnp.zeros((TROW, h_pad), f32).at[:EMB_DIM, :114].set(ew1[24:48])
        wb = jnp.zeros((TROW, h_pad), f32).at[:EMB_DIM, :114].set(ew1[0:24])
        wtrig = (jnp.zeros((16, h_pad), f32)
              .at[0:FOURIER, :114].set(ew1[48:52])
              .at[FOURIER:2 * FOURIER, :114].set(ew1[52:56])
              .at[2 * FOURIER, :114].set(ew1[56]))
        scales = jnp.exp2(-jnp.arange(FOURIER, dtype=f32))
        op = (jnp.zeros((TROW, 16), f32)
              .at[EMB_DIM:EMB_DIM + POS_DIM, 0:FOURIER]
              .set(jnp.tile(scales[None, :], (POS_DIM, 1)))
              .at[EMB_DIM:EMB_DIM + POS_DIM, FOURIER:2 * FOURIER]
              .set(jnp.tile(scales[None, :], (POS_DIM, 1)))
              .at[EMB_DIM:EMB_DIM + POS_DIM, 2 * FOURIER].set(1.0))
        ph = (jnp.zeros((1, 16), f32)
              .at[0, FOURIER:2 * FOURIER].set(jnp.pi / 2))
        eb1 = jnp.zeros((1, h_pad), f32).at[:, :114].set(p['eb1'][None, :])
        ew2 = jnp.zeros((h_pad, M_DIM), f32).at[:114, :].set(p['eW2'])
        eb2 = p['eb2'].reshape(1, M_DIM)
        cw1 = p['cW1']
        cb1 = p['cb1'].reshape(1, M_DIM * 4)
        cw2 = jnp.zeros((M_DIM * 4, 8), f32).at[:, 0:POS_DIM].set(
            jnp.tile(p['cW2'], (1, POS_DIM)))
        cb2 = jnp.zeros((1, 8), f32).at[:, 0:POS_DIM].set(p['cb2'][0])
        nw1a = p['nW1'][0:EMB_DIM]
        nw1b0 = p['nW1'][EMB_DIM:EMB_DIM + PASS_W]
        nw1b1 = p['nW1'][EMB_DIM + PASS_W:]
        nb1 = p['nb1'].reshape(1, EMB_DIM * 2)
        nw2 = p['nW2']
        nb2 = p['nb2'].reshape(1, EMB_DIM)
        ks.append((wa, wb, wtrig, op, ph, eb1, ew2, eb2, cw1, cb1, cw2,
                   cb2, nw1a, nw1b0, nw1b1, nb1, nw2, nb2))

    (fw1, fb1), (fw2, fb2), (fw3, fb3), (fw4, fb4) = params['ffnn']
    w1a, w1b, w1c = fw1[0:24], fw1[24:48], fw1[48:72]
    fb1 = fb1.reshape(1, MLP_DIM)
    fb2 = fb2.reshape(1, MLP_DIM)
    fb3 = fb3.reshape(1, MLP_DIM)
    fb4 = fb4.reshape(1, N_OUT)

    zeros_acc = jnp.zeros((N_NODES, ACC_W), jnp.float32)

    n_grid = N_NODES // NB
    t0 = pl.pallas_call(
        _embed_tc,
        grid=(n_grid,),
        in_specs=[
            pl.BlockSpec((NB, 1), lambda i: (i, 0)),
            pl.BlockSpec((NB, POS_DIM), lambda i: (i, 0)),
            _full_spec((22, EMB_DIM)),
            _full_spec((1, EMB_DIM)),
        ],
        out_specs=pl.BlockSpec((NB, TROW), lambda i: (i, 0)),
        out_shape=jax.ShapeDtypeStruct((N_NODES, TROW), f32),
    )(z2d, pos, wf, bf)

    tables = [t0]
    t_cur = t0
    for (wa, wb, wtrig, op, ph, eb1, ew2, eb2, cw1, cb1, cw2, cb2,
         nw1a, nw1b0, nw1b1, nb1, nw2, nb2) in ks:
        a_rows, b_rows = _sc_gather(t_cur, src, dst)

        e_grid = N_EDGES // EB
        msg = pl.pallas_call(
            _edge_tc,
            grid=(e_grid,),
            in_specs=[
                pl.BlockSpec((EB, TROW), lambda i: (i, 0)),
                pl.BlockSpec((EB, TROW), lambda i: (i, 0)),
                _full_spec((TROW, 128)), _full_spec((TROW, 128)),
                _full_spec((16, 128)), _full_spec((TROW, 16)),
                _full_spec((1, 16)), _full_spec((1, 128)),
                _full_spec((128, M_DIM)), _full_spec((1, M_DIM)),
                _full_spec((M_DIM, M_DIM * 4)), _full_spec((1, M_DIM * 4)),
                _full_spec((M_DIM * 4, 8)), _full_spec((1, 8)),
            ],
            out_specs=pl.BlockSpec((EB, MROW), lambda i: (i, 0)),
            out_shape=jax.ShapeDtypeStruct((N_EDGES, MROW), f32),
        )(a_rows, b_rows, wa, wb, wtrig, op, ph, eb1, ew2, eb2,
          cw1, cb1, cw2, cb2)

        acc0 = _sc_scatter(msg, dst, zeros_acc, 0)
        acc1 = _sc_scatter(msg, dst, zeros_acc, 1)

        t_cur = pl.pallas_call(
            _node_tc,
            grid=(n_grid,),
            in_specs=[
                pl.BlockSpec((NB, TROW), lambda i: (i, 0)),
                pl.BlockSpec((NB, PASS_W), lambda i: (i, 0)),
                pl.BlockSpec((NB, PASS_W), lambda i: (i, 0)),
                _full_spec((EMB_DIM, EMB_DIM * 2)),
                _full_spec((PASS_W, EMB_DIM * 2)),
                _full_spec((M_DIM - PASS_W, EMB_DIM * 2)),
                _full_spec((1, EMB_DIM * 2)),
                _full_spec((EMB_DIM * 2, EMB_DIM)),
                _full_spec((1, EMB_DIM)),
            ],
            out_specs=pl.BlockSpec((NB, TROW), lambda i: (i, 0)),
            out_shape=jax.ShapeDtypeStruct((N_NODES, TROW), f32),
        )(t_cur, acc0, acc1, nw1a, nw1b0, nw1b1, nb1, nw2, nb2)
        tables.append(t_cur)

    out = pl.pallas_call(
        _final_tc,
        grid=(n_grid,),
        in_specs=[
            pl.BlockSpec((NB, TROW), lambda i: (i, 0)),
            pl.BlockSpec((NB, TROW), lambda i: (i, 0)),
            pl.BlockSpec((NB, TROW), lambda i: (i, 0)),
            pl.BlockSpec((NB, 1), lambda i: (i, 0)),
            _full_spec((EMB_DIM, MLP_DIM)), _full_spec((EMB_DIM, MLP_DIM)),
            _full_spec((EMB_DIM, MLP_DIM)), _full_spec((1, MLP_DIM)),
            _full_spec((MLP_DIM, MLP_DIM)), _full_spec((1, MLP_DIM)),
            _full_spec((MLP_DIM, MLP_DIM)), _full_spec((1, MLP_DIM)),
            _full_spec((MLP_DIM, N_OUT)), _full_spec((1, N_OUT)),
        ],
        out_specs=pl.BlockSpec((N_GRAPHS, N_OUT), lambda i: (0, 0)),
        out_shape=jax.ShapeDtypeStruct((N_GRAPHS, N_OUT), f32),
        scratch_shapes=[
            pltpu.VMEM((N_GRAPHS, MLP_DIM), f32),
            pltpu.VMEM((1, N_GRAPHS), f32),
        ],
    )(tables[0], tables[1], tables[2], bid2d,
      w1a, w1b, w1c, fb1, fw2, fb2, fw3, fb3, fw4, fb4)
    return out


# speedup vs baseline: 4.0162x; 1.4087x over previous
import functools

import jax
import jax.numpy as jnp
from jax import lax
from jax.experimental import pallas as pl
from jax.experimental.pallas import tpu as pltpu
from jax.experimental.pallas import tpu_sc as plsc

N_NODES = 50000
N_EDGES = 1600000
N_GRAPHS = 128
POS_DIM = 3
EMB_DIM = 24
M_DIM = 64
MLP_DIM = 256
N_OUT = 4
FOURIER = 4
EDGE_IN = FOURIER * 2 + 1 + EMB_DIM * 2

TROW = 32
MROW = 96
ACC_W = 24
PASS_W = 2 * ACC_W

NC, NS = 2, 16
NW = NC * NS

G_CHUNK = 2000
S_CHUNK = 2000
EB = 2000
NB = 5000


def _sigmoid(x):
    return 0.5 * (jnp.tanh(x * 0.5) + 1.0)


def _silu(x):
    return x * _sigmoid(x)


def _fast_sin(x):
    n = jnp.round(x * (1.0 / (2.0 * jnp.pi)))
    r = x - n * 6.28125 - n * 1.9353071795864769e-3
    pi_s = jnp.where(r > 0.0, jnp.pi, -jnp.pi)
    r = jnp.where(jnp.abs(r) > jnp.pi / 2, pi_s - r, r)
    r2 = r * r
    p = -2.5052108385441720e-8
    p = p * r2 + 2.7557319223985893e-6
    p = p * r2 - 1.9841269841269841e-4
    p = p * r2 + 8.3333333333333333e-3
    p = p * r2 - 1.6666666666666666e-1
    return r + r * r2 * p


def _gather_body(t_hbm, src_hbm, dst_hbm, a_out, b_out, idx_v, rows_v, sem):
    wid = lax.axis_index("s") * NC + lax.axis_index("c")
    n_iters = N_EDGES // (NW * G_CHUNK)
    base_w = wid * (N_EDGES // NW)

    def body(i, carry):
        base = base_w + i * G_CHUNK
        pltpu.sync_copy(src_hbm.at[pl.ds(base, G_CHUNK)], idx_v)
        pltpu.async_copy(t_hbm.at[idx_v], rows_v, sem).wait()
        pltpu.sync_copy(rows_v, a_out.at[pl.ds(base, G_CHUNK)])
        pltpu.sync_copy(dst_hbm.at[pl.ds(base, G_CHUNK)], idx_v)
        pltpu.async_copy(t_hbm.at[idx_v], rows_v, sem).wait()
        pltpu.sync_copy(rows_v, b_out.at[pl.ds(base, G_CHUNK)])
        return carry

    lax.fori_loop(0, n_iters, body, 0)


def _sc_mesh():
    return plsc.VectorSubcoreMesh(
        core_axis_name="c", subcore_axis_name="s",
        num_cores=NC, num_subcores=NS)


def _sc_gather(t, src, dst):
    fn = functools.partial(
        pl.kernel,
        out_type=(
            jax.ShapeDtypeStruct((N_EDGES, TROW), jnp.float32),
            jax.ShapeDtypeStruct((N_EDGES, TROW), jnp.float32),
        ),
        mesh=_sc_mesh(),
        compiler_params=pltpu.CompilerParams(use_tc_tiling_on_sc=False),
        scratch_types=[
            pltpu.VMEM((G_CHUNK,), jnp.int32),
            pltpu.VMEM((G_CHUNK, TROW), jnp.float32),
            pltpu.SemaphoreType.DMA,
        ],
    )(_gather_body)
    return fn(t, src, dst)


def _scatter_body(pass_base, msg_hbm, dst_hbm, zeros_hbm, acc_out,
                  idx_v, msg_v, acc_sh):
    c = lax.axis_index("c")
    s = lax.axis_index("s")
    col0 = pass_base + c * ACC_W

    @pl.when(s == 0)
    def _():
        pltpu.sync_copy(zeros_hbm, acc_sh)

    plsc.subcore_barrier()

    n_iters = N_EDGES // (NS * S_CHUNK)
    base_w = s * (N_EDGES // NS)

    def body(i, carry):
        base = base_w + i * S_CHUNK
        pltpu.sync_copy(dst_hbm.at[pl.ds(base, S_CHUNK)], idx_v)
        pltpu.sync_copy(
            msg_hbm.at[pl.ds(base, S_CHUNK), pl.ds(col0, ACC_W)], msg_v)
        pltpu.sync_copy(msg_v, acc_sh.at[idx_v], add=True)
        return carry

    lax.fori_loop(0, n_iters, body, 0)
    plsc.subcore_barrier()

    @pl.when(s == 0)
    def _():
        pltpu.sync_copy(acc_sh, acc_out.at[:, pl.ds(c * ACC_W, ACC_W)])


def _sc_scatter(msg, dst, zeros_acc, pass_idx):
    fn = functools.partial(
        pl.kernel,
        out_type=jax.ShapeDtypeStruct((N_NODES, PASS_W), jnp.float32),
        mesh=_sc_mesh(),
        compiler_params=pltpu.CompilerParams(use_tc_tiling_on_sc=False),
        scratch_types=[
            pltpu.VMEM((S_CHUNK,), jnp.int32),
            pltpu.VMEM((S_CHUNK, ACC_W), jnp.float32),
            pltpu.VMEM_SHARED((N_NODES, ACC_W), jnp.float32),
        ],
    )(functools.partial(_scatter_body, pass_idx * PASS_W))
    return fn(msg, dst, zeros_acc)


def _embed_tc(z_ref, pos_ref, wf_ref, bf_ref, t_ref):
    z = z_ref[...]
    onehot = (lax.broadcasted_iota(jnp.int32, (NB, 22), 1) == z
              ).astype(jnp.float32)
    feats = onehot @ wf_ref[...] + bf_ref[...]
    t_ref[:, 0:EMB_DIM] = feats
    t_ref[:, EMB_DIM:EMB_DIM + POS_DIM] = pos_ref[...]
    t_ref[:, EMB_DIM + POS_DIM:TROW] = jnp.zeros(
        (NB, TROW - EMB_DIM - POS_DIM), jnp.float32)


def _edge_tc(a_ref, b_ref, wa_ref, wb_ref, wf_ref, op_ref, ph_ref,
             eb1_ref, ew2_ref, eb2_ref, cw1_ref, cb1_ref, cw2_ref, cb2_ref,
             out_ref):
    a = a_ref[...]
    b = b_ref[...]
    dd = a - b
    ang = (dd * dd) @ op_ref[...]
    lane = lax.broadcasted_iota(jnp.int32, (1, 16), 1)
    trig = jnp.where(lane == 2 * FOURIER, ang,
                     _fast_sin(ang + ph_ref[...]))
    h = _silu(a @ wa_ref[...] + b @ wb_ref[...] + trig @ wf_ref[...]
              + eb1_ref[...])
    m = _silu(h @ ew2_ref[...] + eb2_ref[...])
    c1 = _silu(m @ cw1_ref[...] + cb1_ref[...])
    cw8 = c1 @ cw2_ref[...] + cb2_ref[...]
    out_ref[:, 0:M_DIM] = m
    out_ref[:, M_DIM:M_DIM + POS_DIM] = (
        cw8[:, 0:POS_DIM] * dd[:, EMB_DIM:EMB_DIM + POS_DIM])


def _node_tc(t_ref, acc0_ref, acc1_ref, nw1a_ref, nw1b0_ref, nw1b1_ref,
             nb1_ref, nw2_ref, nb2_ref, tout_ref):
    t = t_ref[...]
    acc0 = acc0_ref[...]
    acc1 = acc1_ref[...]
    feats = t[:, 0:EMB_DIM]
    coors = t[:, EMB_DIM:EMB_DIM + POS_DIM]
    mhat = acc1[:, M_DIM - PASS_W:M_DIM - PASS_W + POS_DIM]
    nh = _silu(feats @ nw1a_ref[...] + acc0 @ nw1b0_ref[...]
               + acc1[:, 0:M_DIM - PASS_W] @ nw1b1_ref[...] + nb1_ref[...])
    hid = feats + nh @ nw2_ref[...] + nb2_ref[...]
    tout_ref[:, 0:EMB_DIM] = hid
    tout_ref[:, EMB_DIM:EMB_DIM + POS_DIM] = coors + mhat
    tout_ref[:, EMB_DIM + POS_DIM:TROW] = jnp.zeros(
        (NB, TROW - EMB_DIM - POS_DIM), jnp.float32)


def _final_tc(t0_ref, t1_ref, t2_ref, bid_ref,
              w1a_ref, w1b_ref, w1c_ref, b1_ref, w2_ref, b2_ref,
              w3_ref, b3_ref, w4_ref, b4_ref,
              out_ref, sums_acc, cnts_acc):
    i = pl.program_id(0)
    s0 = _silu(t0_ref[:, 0:EMB_DIM])
    s1 = _silu(t1_ref[:, 0:EMB_DIM])
    s2 = _silu(t2_ref[:, 0:EMB_DIM])
    h = _silu(s0 @ w1a_ref[...] + s1 @ w1b_ref[...] + s2 @ w1c_ref[...]
              + b1_ref[...])
    h = _silu(h @ w2_ref[...] + b2_ref[...])
    h = _silu(h @ w3_ref[...] + b3_ref[...])
    bid = bid_ref[...]
    onehot = (lax.broadcasted_iota(jnp.int32, (NB, N_GRAPHS), 1) == bid
              ).astype(jnp.float32)
    psums = lax.dot_general(onehot, h, (((0,), (0,)), ((), ())))
    pcnts = jnp.sum(onehot, axis=0, keepdims=True)

    @pl.when(i == 0)
    def _():
        sums_acc[...] = jnp.zeros_like(sums_acc)
        cnts_acc[...] = jnp.zeros_like(cnts_acc)

    sums_acc[...] += psums
    cnts_acc[...] += pcnts

    @pl.when(i == pl.num_programs(0) - 1)
    def _():
        cnts = jnp.maximum(cnts_acc[...], 1.0)
        mean = sums_acc[...] / cnts.reshape(N_GRAPHS, 1)
        out_ref[...] = jax.nn.sigmoid(mean @ w4_ref[...] + b4_ref[...])


def _full_spec(shape):
    return pl.BlockSpec(shape, lambda i: tuple(0 for _ in shape))


def kernel(z, pos, edge_index, batch_ids, params):
    f32 = jnp.float32
    src = edge_index[0]
    dst = edge_index[1]
    z2d = z.reshape(N_NODES, 1)
    bid2d = batch_ids.reshape(N_NODES, 1)

    wf = params['emb_table'] @ params['emb_W']
    bf = params['emb_b'].reshape(1, EMB_DIM)

    ks = []
    for p in params['kernels']:
        ew1 = p['eW1']
        h_pad = 128
        wa = jnp.zeros((TROW, h_pad), f32).at[:EMB_DIM, :114].set(ew1[24:48])
        wb = jnp.zeros((TROW, h_pad), f32).at[:EMB_DIM, :114].set(ew1[0:24])
        wtrig = (jnp.zeros((16, h_pad), f32)
              .at[0:FOURIER, :114].set(ew1[48:52])
              .at[FOURIER:2 * FOURIER, :114].set(ew1[52:56])
              .at[2 * FOURIER, :114].set(ew1[56]))
        scales = jnp.exp2(-jnp.arange(FOURIER, dtype=f32))
        op = (jnp.zeros((TROW, 16), f32)
              .at[EMB_DIM:EMB_DIM + POS_DIM, 0:FOURIER]
              .set(jnp.tile(scales[None, :], (POS_DIM, 1)))
              .at[EMB_DIM:EMB_DIM + POS_DIM, FOURIER:2 * FOURIER]
              .set(jnp.tile(scales[None, :], (POS_DIM, 1)))
              .at[EMB_DIM:EMB_DIM + POS_DIM, 2 * FOURIER].set(1.0))
        ph = (jnp.zeros((1, 16), f32)
              .at[0, FOURIER:2 * FOURIER].set(jnp.pi / 2))
        eb1 = jnp.zeros((1, h_pad), f32).at[:, :114].set(p['eb1'][None, :])
        ew2 = jnp.zeros((h_pad, M_DIM), f32).at[:114, :].set(p['eW2'])
        eb2 = p['eb2'].reshape(1, M_DIM)
        cw1 = p['cW1']
        cb1 = p['cb1'].reshape(1, M_DIM * 4)
        cw2 = jnp.zeros((M_DIM * 4, 8), f32).at[:, 0:POS_DIM].set(
            jnp.tile(p['cW2'], (1, POS_DIM)))
        cb2 = jnp.zeros((1, 8), f32).at[:, 0:POS_DIM].set(p['cb2'][0])
        nw1a = p['nW1'][0:EMB_DIM]
        nw1b0 = p['nW1'][EMB_DIM:EMB_DIM + PASS_W]
        nw1b1 = p['nW1'][EMB_DIM + PASS_W:]
        nb1 = p['nb1'].reshape(1, EMB_DIM * 2)
        nw2 = p['nW2']
        nb2 = p['nb2'].reshape(1, EMB_DIM)
        ks.append((wa, wb, wtrig, op, ph, eb1, ew2, eb2, cw1, cb1, cw2,
                   cb2, nw1a, nw1b0, nw1b1, nb1, nw2, nb2))

    (fw1, fb1), (fw2, fb2), (fw3, fb3), (fw4, fb4) = params['ffnn']
    w1a, w1b, w1c = fw1[0:24], fw1[24:48], fw1[48:72]
    fb1 = fb1.reshape(1, MLP_DIM)
    fb2 = fb2.reshape(1, MLP_DIM)
    fb3 = fb3.reshape(1, MLP_DIM)
    fb4 = fb4.reshape(1, N_OUT)

    zeros_acc = jnp.zeros((N_NODES, ACC_W), jnp.float32)

    n_grid = N_NODES // NB
    t0 = pl.pallas_call(
        _embed_tc,
        grid=(n_grid,),
        in_specs=[
            pl.BlockSpec((NB, 1), lambda i: (i, 0)),
            pl.BlockSpec((NB, POS_DIM), lambda i: (i, 0)),
            _full_spec((22, EMB_DIM)),
            _full_spec((1, EMB_DIM)),
        ],
        out_specs=pl.BlockSpec((NB, TROW), lambda i: (i, 0)),
        out_shape=jax.ShapeDtypeStruct((N_NODES, TROW), f32),
    )(z2d, pos, wf, bf)

    tables = [t0]
    t_cur = t0
    for (wa, wb, wtrig, op, ph, eb1, ew2, eb2, cw1, cb1, cw2, cb2,
         nw1a, nw1b0, nw1b1, nb1, nw2, nb2) in ks:
        a_rows, b_rows = _sc_gather(t_cur, src, dst)

        e_grid = N_EDGES // EB
        msg = pl.pallas_call(
            _edge_tc,
            grid=(e_grid,),
            in_specs=[
                pl.BlockSpec((EB, TROW), lambda i: (i, 0)),
                pl.BlockSpec((EB, TROW), lambda i: (i, 0)),
                _full_spec((TROW, 128)), _full_spec((TROW, 128)),
                _full_spec((16, 128)), _full_spec((TROW, 16)),
                _full_spec((1, 16)), _full_spec((1, 128)),
                _full_spec((128, M_DIM)), _full_spec((1, M_DIM)),
                _full_spec((M_DIM, M_DIM * 4)), _full_spec((1, M_DIM * 4)),
                _full_spec((M_DIM * 4, 8)), _full_spec((1, 8)),
            ],
            out_specs=pl.BlockSpec((EB, MROW), lambda i: (i, 0)),
            out_shape=jax.ShapeDtypeStruct((N_EDGES, MROW), f32),
        )(a_rows, b_rows, wa, wb, wtrig, op, ph, eb1, ew2, eb2,
          cw1, cb1, cw2, cb2)

        acc0 = _sc_scatter(msg, dst, zeros_acc, 0)
        acc1 = _sc_scatter(msg, dst, zeros_acc, 1)

        t_cur = pl.pallas_call(
            _node_tc,
            grid=(n_grid,),
            in_specs=[
                pl.BlockSpec((NB, TROW), lambda i: (i, 0)),
                pl.BlockSpec((NB, PASS_W), lambda i: (i, 0)),
                pl.BlockSpec((NB, PASS_W), lambda i: (i, 0)),
                _full_spec((EMB_DIM, EMB_DIM * 2)),
                _full_spec((PASS_W, EMB_DIM * 2)),
                _full_spec((M_DIM - PASS_W, EMB_DIM * 2)),
                _full_spec((1, EMB_DIM * 2)),
                _full_spec((EMB_DIM * 2, EMB_DIM)),
                _full_spec((1, EMB_DIM)),
            ],
            out_specs=pl.BlockSpec((NB, TROW), lambda i: (i, 0)),
            out_shape=jax.ShapeDtypeStruct((N_NODES, TROW), f32),
        )(t_cur, acc0, acc1, nw1a, nw1b0, nw1b1, nb1, nw2, nb2)
        tables.append(t_cur)

    out = pl.pallas_call(
        _final_tc,
        grid=(n_grid,),
        in_specs=[
            pl.BlockSpec((NB, TROW), lambda i: (i, 0)),
            pl.BlockSpec((NB, TROW), lambda i: (i, 0)),
            pl.BlockSpec((NB, TROW), lambda i: (i, 0)),
            pl.BlockSpec((NB, 1), lambda i: (i, 0)),
            _full_spec((EMB_DIM, MLP_DIM)), _full_spec((EMB_DIM, MLP_DIM)),
            _full_spec((EMB_DIM, MLP_DIM)), _full_spec((1, MLP_DIM)),
            _full_spec((MLP_DIM, MLP_DIM)), _full_spec((1, MLP_DIM)),
            _full_spec((MLP_DIM, MLP_DIM)), _full_spec((1, MLP_DIM)),
            _full_spec((MLP_DIM, N_OUT)), _full_spec((1, N_OUT)),
        ],
        out_specs=pl.BlockSpec((N_GRAPHS, N_OUT), lambda i: (0, 0)),
        out_shape=jax.ShapeDtypeStruct((N_GRAPHS, N_OUT), f32),
        scratch_shapes=[
            pltpu.VMEM((N_GRAPHS, MLP_DIM), f32),
            pltpu.VMEM((1, N_GRAPHS), f32),
        ],
    )(tables[0], tables[1], tables[2], bid2d,
      w1a, w1b, w1c, fb1, fw2, fb2, fw3, fb3, fw4, fb4)
    return out


# speedup vs baseline: 4.9014x; 1.2204x over previous
import functools

import jax
import jax.numpy as jnp
from jax import lax
from jax.experimental import pallas as pl
from jax.experimental.pallas import tpu as pltpu
from jax.experimental.pallas import tpu_sc as plsc

N_NODES = 50000
N_EDGES = 1600000
N_GRAPHS = 128
POS_DIM = 3
EMB_DIM = 24
M_DIM = 64
MLP_DIM = 256
N_OUT = 4
FOURIER = 4
EDGE_IN = FOURIER * 2 + 1 + EMB_DIM * 2

TROW = 32
MROW = 128
ACC_W = 24
PASS_W = 2 * ACC_W

NC, NS = 2, 16
NW = NC * NS

G_CHUNK = 2000
S_CHUNK = 2000
EB = 2000
NB = 5000


def _sigmoid(x):
    return 0.5 * (jnp.tanh(x * 0.5) + 1.0)


def _silu(x):
    return x * _sigmoid(x)


def _fast_sin(x):
    n = jnp.round(x * (1.0 / (2.0 * jnp.pi)))
    r = x - n * 6.28125 - n * 1.9353071795864769e-3
    pi_s = jnp.where(r > 0.0, jnp.pi, -jnp.pi)
    r = jnp.where(jnp.abs(r) > jnp.pi / 2, pi_s - r, r)
    r2 = r * r
    p = -2.5052108385441720e-8
    p = p * r2 + 2.7557319223985893e-6
    p = p * r2 - 1.9841269841269841e-4
    p = p * r2 + 8.3333333333333333e-3
    p = p * r2 - 1.6666666666666666e-1
    return r + r * r2 * p


def _gather_body(t_hbm, src_hbm, dst_hbm, a_out, b_out, idx_v, rows_v, sem):
    wid = lax.axis_index("s") * NC + lax.axis_index("c")
    n_iters = N_EDGES // (NW * G_CHUNK)
    base_w = wid * (N_EDGES // NW)

    def body(i, carry):
        base = base_w + i * G_CHUNK
        pltpu.sync_copy(src_hbm.at[pl.ds(base, G_CHUNK)], idx_v)
        pltpu.async_copy(t_hbm.at[idx_v], rows_v, sem).wait()
        pltpu.sync_copy(rows_v, a_out.at[pl.ds(base, G_CHUNK)])
        pltpu.sync_copy(dst_hbm.at[pl.ds(base, G_CHUNK)], idx_v)
        pltpu.async_copy(t_hbm.at[idx_v], rows_v, sem).wait()
        pltpu.sync_copy(rows_v, b_out.at[pl.ds(base, G_CHUNK)])
        return carry

    lax.fori_loop(0, n_iters, body, 0)


def _sc_mesh():
    return plsc.VectorSubcoreMesh(
        core_axis_name="c", subcore_axis_name="s",
        num_cores=NC, num_subcores=NS)


def _sc_gather(t, src, dst):
    fn = functools.partial(
        pl.kernel,
        out_type=(
            jax.ShapeDtypeStruct((N_EDGES, TROW), jnp.float32),
            jax.ShapeDtypeStruct((N_EDGES, TROW), jnp.float32),
        ),
        mesh=_sc_mesh(),
        compiler_params=pltpu.CompilerParams(use_tc_tiling_on_sc=False),
        scratch_types=[
            pltpu.VMEM((G_CHUNK,), jnp.int32),
            pltpu.VMEM((G_CHUNK, TROW), jnp.float32),
            pltpu.SemaphoreType.DMA,
        ],
    )(_gather_body)
    return fn(t, src, dst)


def _scatter_body(pass_base, msg_hbm, dst_hbm, zeros_hbm, acc_out,
                  idx_v, msg_v, acc_sh):
    c = lax.axis_index("c")
    s = lax.axis_index("s")
    col0 = pass_base + c * ACC_W

    @pl.when(s == 0)
    def _():
        pltpu.sync_copy(zeros_hbm, acc_sh)

    plsc.subcore_barrier()

    n_iters = N_EDGES // (NS * S_CHUNK)
    base_w = s * (N_EDGES // NS)

    def body(i, carry):
        base = base_w + i * S_CHUNK
        pltpu.sync_copy(dst_hbm.at[pl.ds(base, S_CHUNK)], idx_v)
        pltpu.sync_copy(
            msg_hbm.at[pl.ds(base, S_CHUNK), pl.ds(col0, ACC_W)], msg_v)
        pltpu.sync_copy(msg_v, acc_sh.at[idx_v], add=True)
        return carry

    lax.fori_loop(0, n_iters, body, 0)
    plsc.subcore_barrier()

    @pl.when(s == 0)
    def _():
        pltpu.sync_copy(acc_sh, acc_out.at[:, pl.ds(c * ACC_W, ACC_W)])


def _sc_scatter(msg, dst, zeros_acc, pass_idx):
    fn = functools.partial(
        pl.kernel,
        out_type=jax.ShapeDtypeStruct((N_NODES, PASS_W), jnp.float32),
        mesh=_sc_mesh(),
        compiler_params=pltpu.CompilerParams(use_tc_tiling_on_sc=False),
        scratch_types=[
            pltpu.VMEM((S_CHUNK,), jnp.int32),
            pltpu.VMEM((S_CHUNK, ACC_W), jnp.float32),
            pltpu.VMEM_SHARED((N_NODES, ACC_W), jnp.float32),
        ],
    )(functools.partial(_scatter_body, pass_idx * PASS_W))
    return fn(msg, dst, zeros_acc)


def _embed_tc(z_ref, pos_ref, wf_ref, bf_ref, t_ref):
    z = z_ref[...]
    onehot = (lax.broadcasted_iota(jnp.int32, (NB, 22), 1) == z
              ).astype(jnp.float32)
    feats = onehot @ wf_ref[...] + bf_ref[...]
    t_ref[:, 0:EMB_DIM] = feats
    t_ref[:, EMB_DIM:EMB_DIM + POS_DIM] = pos_ref[...]
    t_ref[:, EMB_DIM + POS_DIM:TROW] = jnp.zeros(
        (NB, TROW - EMB_DIM - POS_DIM), jnp.float32)


def _edge_tc(a_ref, b_ref, wa_ref, wb_ref, wf_ref, op_ref, ph_ref,
             eb1_ref, ew2_ref, eb2_ref, cw1_ref, cb1_ref, cw2_ref, cb2_ref,
             out_ref):
    a = a_ref[...]
    b = b_ref[...]
    dd = a - b
    ang = (dd * dd) @ op_ref[...]
    lane = lax.broadcasted_iota(jnp.int32, (1, 16), 1)
    trig = jnp.where(lane == 2 * FOURIER, ang,
                     _fast_sin(ang + ph_ref[...]))
    h = _silu(a @ wa_ref[...] + b @ wb_ref[...] + trig @ wf_ref[...]
              + eb1_ref[...])
    m = _silu(h @ ew2_ref[...] + eb2_ref[...])
    c1 = _silu(m @ cw1_ref[...] + cb1_ref[...])
    cw8 = c1 @ cw2_ref[...] + cb2_ref[...]
    out_ref[:, 0:M_DIM] = m
    out_ref[:, M_DIM:M_DIM + POS_DIM] = (
        cw8[:, 0:POS_DIM] * dd[:, EMB_DIM:EMB_DIM + POS_DIM])


def _node_tc(t_ref, acc0_ref, acc1_ref, nw1a_ref, nw1b0_ref, nw1b1_ref,
             nb1_ref, nw2_ref, nb2_ref, tout_ref):
    t = t_ref[...]
    acc0 = acc0_ref[...]
    acc1 = acc1_ref[...]
    feats = t[:, 0:EMB_DIM]
    coors = t[:, EMB_DIM:EMB_DIM + POS_DIM]
    mhat = acc1[:, M_DIM - PASS_W:M_DIM - PASS_W + POS_DIM]
    nh = _silu(feats @ nw1a_ref[...] + acc0 @ nw1b0_ref[...]
               + acc1[:, 0:M_DIM - PASS_W] @ nw1b1_ref[...] + nb1_ref[...])
    hid = feats + nh @ nw2_ref[...] + nb2_ref[...]
    tout_ref[:, 0:EMB_DIM] = hid
    tout_ref[:, EMB_DIM:EMB_DIM + POS_DIM] = coors + mhat
    tout_ref[:, EMB_DIM + POS_DIM:TROW] = jnp.zeros(
        (NB, TROW - EMB_DIM - POS_DIM), jnp.float32)


def _final_tc(t0_ref, t1_ref, t2_ref, bid_ref,
              w1a_ref, w1b_ref, w1c_ref, b1_ref, w2_ref, b2_ref,
              w3_ref, b3_ref, w4_ref, b4_ref,
              out_ref, sums_acc, cnts_acc):
    i = pl.program_id(0)
    s0 = _silu(t0_ref[:, 0:EMB_DIM])
    s1 = _silu(t1_ref[:, 0:EMB_DIM])
    s2 = _silu(t2_ref[:, 0:EMB_DIM])
    h = _silu(s0 @ w1a_ref[...] + s1 @ w1b_ref[...] + s2 @ w1c_ref[...]
              + b1_ref[...])
    h = _silu(h @ w2_ref[...] + b2_ref[...])
    h = _silu(h @ w3_ref[...] + b3_ref[...])
    bid = bid_ref[...]
    onehot = (lax.broadcasted_iota(jnp.int32, (NB, N_GRAPHS), 1) == bid
              ).astype(jnp.float32)
    psums = lax.dot_general(onehot, h, (((0,), (0,)), ((), ())))
    pcnts = jnp.sum(onehot, axis=0, keepdims=True)

    @pl.when(i == 0)
    def _():
        sums_acc[...] = jnp.zeros_like(sums_acc)
        cnts_acc[...] = jnp.zeros_like(cnts_acc)

    sums_acc[...] += psums
    cnts_acc[...] += pcnts

    @pl.when(i == pl.num_programs(0) - 1)
    def _():
        cnts = jnp.maximum(cnts_acc[...], 1.0)
        mean = sums_acc[...] / cnts.reshape(N_GRAPHS, 1)
        out_ref[...] = jax.nn.sigmoid(mean @ w4_ref[...] + b4_ref[...])


def _full_spec(shape):
    return pl.BlockSpec(shape, lambda i: tuple(0 for _ in shape))


def kernel(z, pos, edge_index, batch_ids, params):
    f32 = jnp.float32
    src = edge_index[0]
    dst = edge_index[1]
    z2d = z.reshape(N_NODES, 1)
    bid2d = batch_ids.reshape(N_NODES, 1)

    wf = params['emb_table'] @ params['emb_W']
    bf = params['emb_b'].reshape(1, EMB_DIM)

    ks = []
    for p in params['kernels']:
        ew1 = p['eW1']
        h_pad = 128
        wa = jnp.zeros((TROW, h_pad), f32).at[:EMB_DIM, :114].set(ew1[24:48])
        wb = jnp.zeros((TROW, h_pad), f32).at[:EMB_DIM, :114].set(ew1[0:24])
        wtrig = (jnp.zeros((16, h_pad), f32)
              .at[0:FOURIER, :114].set(ew1[48:52])
              .at[FOURIER:2 * FOURIER, :114].set(ew1[52:56])
              .at[2 * FOURIER, :114].set(ew1[56]))
        scales = jnp.exp2(-jnp.arange(FOURIER, dtype=f32))
        op = (jnp.zeros((TROW, 16), f32)
              .at[EMB_DIM:EMB_DIM + POS_DIM, 0:FOURIER]
              .set(jnp.tile(scales[None, :], (POS_DIM, 1)))
              .at[EMB_DIM:EMB_DIM + POS_DIM, FOURIER:2 * FOURIER]
              .set(jnp.tile(scales[None, :], (POS_DIM, 1)))
              .at[EMB_DIM:EMB_DIM + POS_DIM, 2 * FOURIER].set(1.0))
        ph = (jnp.zeros((1, 16), f32)
              .at[0, FOURIER:2 * FOURIER].set(jnp.pi / 2))
        eb1 = jnp.zeros((1, h_pad), f32).at[:, :114].set(p['eb1'][None, :])
        ew2 = jnp.zeros((h_pad, M_DIM), f32).at[:114, :].set(p['eW2'])
        eb2 = p['eb2'].reshape(1, M_DIM)
        cw1 = p['cW1']
        cb1 = p['cb1'].reshape(1, M_DIM * 4)
        cw2 = jnp.zeros((M_DIM * 4, 8), f32).at[:, 0:POS_DIM].set(
            jnp.tile(p['cW2'], (1, POS_DIM)))
        cb2 = jnp.zeros((1, 8), f32).at[:, 0:POS_DIM].set(p['cb2'][0])
        nw1a = p['nW1'][0:EMB_DIM]
        nw1b0 = p['nW1'][EMB_DIM:EMB_DIM + PASS_W]
        nw1b1 = p['nW1'][EMB_DIM + PASS_W:]
        nb1 = p['nb1'].reshape(1, EMB_DIM * 2)
        nw2 = p['nW2']
        nb2 = p['nb2'].reshape(1, EMB_DIM)
        ks.append((wa, wb, wtrig, op, ph, eb1, ew2, eb2, cw1, cb1, cw2,
                   cb2, nw1a, nw1b0, nw1b1, nb1, nw2, nb2))

    (fw1, fb1), (fw2, fb2), (fw3, fb3), (fw4, fb4) = params['ffnn']
    w1a, w1b, w1c = fw1[0:24], fw1[24:48], fw1[48:72]
    fb1 = fb1.reshape(1, MLP_DIM)
    fb2 = fb2.reshape(1, MLP_DIM)
    fb3 = fb3.reshape(1, MLP_DIM)
    fb4 = fb4.reshape(1, N_OUT)

    zeros_acc = jnp.zeros((N_NODES, ACC_W), jnp.float32)

    n_grid = N_NODES // NB
    t0 = pl.pallas_call(
        _embed_tc,
        grid=(n_grid,),
        in_specs=[
            pl.BlockSpec((NB, 1), lambda i: (i, 0)),
            pl.BlockSpec((NB, POS_DIM), lambda i: (i, 0)),
            _full_spec((22, EMB_DIM)),
            _full_spec((1, EMB_DIM)),
        ],
        out_specs=pl.BlockSpec((NB, TROW), lambda i: (i, 0)),
        out_shape=jax.ShapeDtypeStruct((N_NODES, TROW), f32),
    )(z2d, pos, wf, bf)

    tables = [t0]
    t_cur = t0
    for (wa, wb, wtrig, op, ph, eb1, ew2, eb2, cw1, cb1, cw2, cb2,
         nw1a, nw1b0, nw1b1, nb1, nw2, nb2) in ks:
        a_rows, b_rows = _sc_gather(t_cur, src, dst)

        e_grid = N_EDGES // EB
        msg = pl.pallas_call(
            _edge_tc,
            grid=(e_grid,),
            in_specs=[
                pl.BlockSpec((EB, TROW), lambda i: (i, 0)),
                pl.BlockSpec((EB, TROW), lambda i: (i, 0)),
                _full_spec((TROW, 128)), _full_spec((TROW, 128)),
                _full_spec((16, 128)), _full_spec((TROW, 16)),
                _full_spec((1, 16)), _full_spec((1, 128)),
                _full_spec((128, M_DIM)), _full_spec((1, M_DIM)),
                _full_spec((M_DIM, M_DIM * 4)), _full_spec((1, M_DIM * 4)),
                _full_spec((M_DIM * 4, 8)), _full_spec((1, 8)),
            ],
            out_specs=pl.BlockSpec((EB, MROW), lambda i: (i, 0)),
            out_shape=jax.ShapeDtypeStruct((N_EDGES, MROW), f32),
        )(a_rows, b_rows, wa, wb, wtrig, op, ph, eb1, ew2, eb2,
          cw1, cb1, cw2, cb2)

        acc0 = _sc_scatter(msg, dst, zeros_acc, 0)
        acc1 = _sc_scatter(msg, dst, zeros_acc, 1)

        t_cur = pl.pallas_call(
            _node_tc,
            grid=(n_grid,),
            in_specs=[
                pl.BlockSpec((NB, TROW), lambda i: (i, 0)),
                pl.BlockSpec((NB, PASS_W), lambda i: (i, 0)),
                pl.BlockSpec((NB, PASS_W), lambda i: (i, 0)),
                _full_spec((EMB_DIM, EMB_DIM * 2)),
                _full_spec((PASS_W, EMB_DIM * 2)),
                _full_spec((M_DIM - PASS_W, EMB_DIM * 2)),
                _full_spec((1, EMB_DIM * 2)),
                _full_spec((EMB_DIM * 2, EMB_DIM)),
                _full_spec((1, EMB_DIM)),
            ],
            out_specs=pl.BlockSpec((NB, TROW), lambda i: (i, 0)),
            out_shape=jax.ShapeDtypeStruct((N_NODES, TROW), f32),
        )(t_cur, acc0, acc1, nw1a, nw1b0, nw1b1, nb1, nw2, nb2)
        tables.append(t_cur)

    out = pl.pallas_call(
        _final_tc,
        grid=(n_grid,),
        in_specs=[
            pl.BlockSpec((NB, TROW), lambda i: (i, 0)),
            pl.BlockSpec((NB, TROW), lambda i: (i, 0)),
            pl.BlockSpec((NB, TROW), lambda i: (i, 0)),
            pl.BlockSpec((NB, 1), lambda i: (i, 0)),
            _full_spec((EMB_DIM, MLP_DIM)), _full_spec((EMB_DIM, MLP_DIM)),
            _full_spec((EMB_DIM, MLP_DIM)), _full_spec((1, MLP_DIM)),
            _full_spec((MLP_DIM, MLP_DIM)), _full_spec((1, MLP_DIM)),
            _full_spec((MLP_DIM, MLP_DIM)), _full_spec((1, MLP_DIM)),
            _full_spec((MLP_DIM, N_OUT)), _full_spec((1, N_OUT)),
        ],
        out_specs=pl.BlockSpec((N_GRAPHS, N_OUT), lambda i: (0, 0)),
        out_shape=jax.ShapeDtypeStruct((N_GRAPHS, N_OUT), f32),
        scratch_shapes=[
            pltpu.VMEM((N_GRAPHS, MLP_DIM), f32),
            pltpu.VMEM((1, N_GRAPHS), f32),
        ],
    )(tables[0], tables[1], tables[2], bid2d,
      w1a, w1b, w1c, fb1, fw2, fb2, fw3, fb3, fw4, fb4)
    return out


# speedup vs baseline: 5.6557x; 1.1539x over previous
import functools

import jax
import jax.numpy as jnp
from jax import lax
from jax.experimental import pallas as pl
from jax.experimental.pallas import tpu as pltpu
from jax.experimental.pallas import tpu_sc as plsc

N_NODES = 50000
N_EDGES = 1600000
N_GRAPHS = 128
POS_DIM = 3
EMB_DIM = 24
M_DIM = 64
MLP_DIM = 256
N_OUT = 4
FOURIER = 4
EDGE_IN = FOURIER * 2 + 1 + EMB_DIM * 2

TROW = 32
MROW = 128
ACC_W = 24
PASS_W = 2 * ACC_W

NC, NS = 2, 16
NW = NC * NS

PROW = 128
RELO = 120
G_CHUNK = 1000
S_CHUNK = 2000
EB = 2000
NB = 5000


def _sigmoid(x):
    return 0.5 * (jnp.tanh(x * 0.5) + 1.0)


def _silu(x):
    return x * _sigmoid(x)


def _fast_sin(x):
    n = jnp.round(x * (1.0 / (2.0 * jnp.pi)))
    r = x - n * 6.28125 - n * 1.9353071795864769e-3
    pi_s = jnp.where(r > 0.0, jnp.pi, -jnp.pi)
    r = jnp.where(jnp.abs(r) > jnp.pi / 2, pi_s - r, r)
    r2 = r * r
    p = -2.5052108385441720e-8
    p = p * r2 + 2.7557319223985893e-6
    p = p * r2 - 1.9841269841269841e-4
    p = p * r2 + 8.3333333333333333e-3
    p = p * r2 - 1.6666666666666666e-1
    return r + r * r2 * p


def _gather_body(pa_hbm, pb_hbm, src_hbm, dst_hbm, h_out, idx_v, rows_v, sem):
    wid = lax.axis_index("s") * NC + lax.axis_index("c")
    n_iters = N_EDGES // (NW * G_CHUNK)
    base_w = wid * (N_EDGES // NW)

    def body(i, carry):
        base = base_w + i * G_CHUNK
        pltpu.sync_copy(src_hbm.at[pl.ds(base, G_CHUNK)], idx_v)
        pltpu.async_copy(pa_hbm.at[idx_v], rows_v, sem).wait()
        pltpu.sync_copy(dst_hbm.at[pl.ds(base, G_CHUNK)], idx_v)
        pltpu.async_copy(pb_hbm.at[idx_v], rows_v, sem, add=True).wait()
        pltpu.sync_copy(rows_v, h_out.at[pl.ds(base, G_CHUNK)])
        return carry

    lax.fori_loop(0, n_iters, body, 0)


def _sc_mesh():
    return plsc.VectorSubcoreMesh(
        core_axis_name="c", subcore_axis_name="s",
        num_cores=NC, num_subcores=NS)


def _sc_gather(pa, pb, src, dst):
    fn = functools.partial(
        pl.kernel,
        out_type=jax.ShapeDtypeStruct((N_EDGES, PROW), jnp.float32),
        mesh=_sc_mesh(),
        compiler_params=pltpu.CompilerParams(use_tc_tiling_on_sc=False),
        scratch_types=[
            pltpu.VMEM((G_CHUNK,), jnp.int32),
            pltpu.VMEM((G_CHUNK, PROW), jnp.float32),
            pltpu.SemaphoreType.DMA,
        ],
    )(_gather_body)
    return fn(pa, pb, src, dst)


def _scatter_body(pass_base, msg_hbm, dst_hbm, zeros_hbm, acc_out,
                  idx_v, msg_v, acc_sh):
    c = lax.axis_index("c")
    s = lax.axis_index("s")
    col0 = pass_base + c * ACC_W

    @pl.when(s == 0)
    def _():
        pltpu.sync_copy(zeros_hbm, acc_sh)

    plsc.subcore_barrier()

    n_iters = N_EDGES // (NS * S_CHUNK)
    base_w = s * (N_EDGES // NS)

    def body(i, carry):
        base = base_w + i * S_CHUNK
        pltpu.sync_copy(dst_hbm.at[pl.ds(base, S_CHUNK)], idx_v)
        pltpu.sync_copy(
            msg_hbm.at[pl.ds(base, S_CHUNK), pl.ds(col0, ACC_W)], msg_v)
        pltpu.sync_copy(msg_v, acc_sh.at[idx_v], add=True)
        return carry

    lax.fori_loop(0, n_iters, body, 0)
    plsc.subcore_barrier()

    @pl.when(s == 0)
    def _():
        pltpu.sync_copy(acc_sh, acc_out.at[:, pl.ds(c * ACC_W, ACC_W)])


def _sc_scatter(msg, dst, zeros_acc, pass_idx):
    fn = functools.partial(
        pl.kernel,
        out_type=jax.ShapeDtypeStruct((N_NODES, PASS_W), jnp.float32),
        mesh=_sc_mesh(),
        compiler_params=pltpu.CompilerParams(use_tc_tiling_on_sc=False),
        scratch_types=[
            pltpu.VMEM((S_CHUNK,), jnp.int32),
            pltpu.VMEM((S_CHUNK, ACC_W), jnp.float32),
            pltpu.VMEM_SHARED((N_NODES, ACC_W), jnp.float32),
        ],
    )(functools.partial(_scatter_body, pass_idx * PASS_W))
    return fn(msg, dst, zeros_acc)


def _write_proj(pa_ref, pb_ref, feats, coors, wa_ref, wb_ref):
    pa_ref[...] = feats @ wa_ref[...]
    pa_ref[:, RELO:RELO + POS_DIM] = coors
    pb_ref[...] = feats @ wb_ref[...]
    pb_ref[:, RELO:RELO + POS_DIM] = -coors


def _embed_tc(z_ref, pos_ref, wf_ref, bf_ref, wa_ref, wb_ref,
              t_ref, pa_ref, pb_ref):
    z = z_ref[...]
    onehot = (lax.broadcasted_iota(jnp.int32, (NB, 22), 1) == z
              ).astype(jnp.float32)
    feats = onehot @ wf_ref[...] + bf_ref[...]
    coors = pos_ref[...]
    t_ref[:, 0:EMB_DIM] = feats
    t_ref[:, EMB_DIM:EMB_DIM + POS_DIM] = coors
    t_ref[:, EMB_DIM + POS_DIM:TROW] = jnp.zeros(
        (NB, TROW - EMB_DIM - POS_DIM), jnp.float32)
    _write_proj(pa_ref, pb_ref, feats, coors, wa_ref, wb_ref)


def _edge_tc(h0_ref, wf_ref, op_ref, ph_ref,
             eb1_ref, ew2_ref, eb2_ref, cw1_ref, cb1_ref, cw2_ref, cb2_ref,
             out_ref):
    h0 = h0_ref[...]
    ang = (h0 * h0) @ op_ref[...]
    lane = lax.broadcasted_iota(jnp.int32, (1, 16), 1)
    trig = jnp.where(lane == 2 * FOURIER, ang,
                     _fast_sin(ang + ph_ref[...]))
    h = _silu(h0 + trig @ wf_ref[...] + eb1_ref[...])
    m = _silu(h @ ew2_ref[...] + eb2_ref[...])
    c1 = _silu(m @ cw1_ref[...] + cb1_ref[...])
    cw8 = c1 @ cw2_ref[...] + cb2_ref[...]
    out_ref[:, 0:M_DIM] = m
    out_ref[:, M_DIM:M_DIM + POS_DIM] = (
        cw8[:, 0:POS_DIM] * h0[:, RELO:RELO + POS_DIM])


def _node_tc(t_ref, acc0_ref, acc1_ref, nw1a_ref, nw1b0_ref, nw1b1_ref,
             nb1_ref, nw2_ref, nb2_ref, wa_ref, wb_ref,
             tout_ref, pa_ref, pb_ref):
    t = t_ref[...]
    acc0 = acc0_ref[...]
    acc1 = acc1_ref[...]
    feats = t[:, 0:EMB_DIM]
    coors = t[:, EMB_DIM:EMB_DIM + POS_DIM]
    mhat = acc1[:, M_DIM - PASS_W:M_DIM - PASS_W + POS_DIM]
    nh = _silu(feats @ nw1a_ref[...] + acc0 @ nw1b0_ref[...]
               + acc1[:, 0:M_DIM - PASS_W] @ nw1b1_ref[...] + nb1_ref[...])
    hid = feats + nh @ nw2_ref[...] + nb2_ref[...]
    coors_out = coors + mhat
    tout_ref[:, 0:EMB_DIM] = hid
    tout_ref[:, EMB_DIM:EMB_DIM + POS_DIM] = coors_out
    tout_ref[:, EMB_DIM + POS_DIM:TROW] = jnp.zeros(
        (NB, TROW - EMB_DIM - POS_DIM), jnp.float32)
    _write_proj(pa_ref, pb_ref, hid, coors_out, wa_ref, wb_ref)


def _final_tc(t0_ref, t1_ref, t2_ref, bid_ref,
              w1a_ref, w1b_ref, w1c_ref, b1_ref, w2_ref, b2_ref,
              w3_ref, b3_ref, w4_ref, b4_ref,
              out_ref, sums_acc, cnts_acc):
    i = pl.program_id(0)
    s0 = _silu(t0_ref[:, 0:EMB_DIM])
    s1 = _silu(t1_ref[:, 0:EMB_DIM])
    s2 = _silu(t2_ref[:, 0:EMB_DIM])
    h = _silu(s0 @ w1a_ref[...] + s1 @ w1b_ref[...] + s2 @ w1c_ref[...]
              + b1_ref[...])
    h = _silu(h @ w2_ref[...] + b2_ref[...])
    h = _silu(h @ w3_ref[...] + b3_ref[...])
    bid = bid_ref[...]
    onehot = (lax.broadcasted_iota(jnp.int32, (NB, N_GRAPHS), 1) == bid
              ).astype(jnp.float32)
    psums = lax.dot_general(onehot, h, (((0,), (0,)), ((), ())))
    pcnts = jnp.sum(onehot, axis=0, keepdims=True)

    @pl.when(i == 0)
    def _():
        sums_acc[...] = jnp.zeros_like(sums_acc)
        cnts_acc[...] = jnp.zeros_like(cnts_acc)

    sums_acc[...] += psums
    cnts_acc[...] += pcnts

    @pl.when(i == pl.num_programs(0) - 1)
    def _():
        cnts = jnp.maximum(cnts_acc[...], 1.0)
        mean = sums_acc[...] / cnts.reshape(N_GRAPHS, 1)
        out_ref[...] = jax.nn.sigmoid(mean @ w4_ref[...] + b4_ref[...])


def _full_spec(shape):
    return pl.BlockSpec(shape, lambda i: tuple(0 for _ in shape))


def kernel(z, pos, edge_index, batch_ids, params):
    f32 = jnp.float32
    src = edge_index[0]
    dst = edge_index[1]
    z2d = z.reshape(N_NODES, 1)
    bid2d = batch_ids.reshape(N_NODES, 1)

    wf = params['emb_table'] @ params['emb_W']
    bf = params['emb_b'].reshape(1, EMB_DIM)

    ks = []
    for p in params['kernels']:
        ew1 = p['eW1']
        h_pad = 128
        wa = jnp.zeros((EMB_DIM, h_pad), f32).at[:, :114].set(ew1[24:48])
        wb = jnp.zeros((EMB_DIM, h_pad), f32).at[:, :114].set(ew1[0:24])
        wtrig = (jnp.zeros((16, h_pad), f32)
              .at[0:FOURIER, :114].set(ew1[48:52])
              .at[FOURIER:2 * FOURIER, :114].set(ew1[52:56])
              .at[2 * FOURIER, :114].set(ew1[56]))
        scales = jnp.exp2(-jnp.arange(FOURIER, dtype=f32))
        op = (jnp.zeros((PROW, 16), f32)
              .at[RELO:RELO + POS_DIM, 0:FOURIER]
              .set(jnp.tile(scales[None, :], (POS_DIM, 1)))
              .at[RELO:RELO + POS_DIM, FOURIER:2 * FOURIER]
              .set(jnp.tile(scales[None, :], (POS_DIM, 1)))
              .at[RELO:RELO + POS_DIM, 2 * FOURIER].set(1.0))
        ph = (jnp.zeros((1, 16), f32)
              .at[0, FOURIER:2 * FOURIER].set(jnp.pi / 2))
        eb1 = jnp.zeros((1, h_pad), f32).at[:, :114].set(p['eb1'][None, :])
        ew2 = jnp.zeros((h_pad, M_DIM), f32).at[:114, :].set(p['eW2'])
        eb2 = p['eb2'].reshape(1, M_DIM)
        cw1 = p['cW1']
        cb1 = p['cb1'].reshape(1, M_DIM * 4)
        cw2 = jnp.zeros((M_DIM * 4, 8), f32).at[:, 0:POS_DIM].set(
            jnp.tile(p['cW2'], (1, POS_DIM)))
        cb2 = jnp.zeros((1, 8), f32).at[:, 0:POS_DIM].set(p['cb2'][0])
        nw1a = p['nW1'][0:EMB_DIM]
        nw1b0 = p['nW1'][EMB_DIM:EMB_DIM + PASS_W]
        nw1b1 = p['nW1'][EMB_DIM + PASS_W:]
        nb1 = p['nb1'].reshape(1, EMB_DIM * 2)
        nw2 = p['nW2']
        nb2 = p['nb2'].reshape(1, EMB_DIM)
        ks.append((wa, wb, wtrig, op, ph, eb1, ew2, eb2, cw1, cb1, cw2,
                   cb2, nw1a, nw1b0, nw1b1, nb1, nw2, nb2))

    (fw1, fb1), (fw2, fb2), (fw3, fb3), (fw4, fb4) = params['ffnn']
    w1a, w1b, w1c = fw1[0:24], fw1[24:48], fw1[48:72]
    fb1 = fb1.reshape(1, MLP_DIM)
    fb2 = fb2.reshape(1, MLP_DIM)
    fb3 = fb3.reshape(1, MLP_DIM)
    fb4 = fb4.reshape(1, N_OUT)

    zeros_acc = jnp.zeros((N_NODES, ACC_W), jnp.float32)

    n_grid = N_NODES // NB
    nspec = pl.BlockSpec((NB, TROW), lambda i: (i, 0))
    pspec = pl.BlockSpec((NB, PROW), lambda i: (i, 0))
    t0, pa, pb = pl.pallas_call(
        _embed_tc,
        grid=(n_grid,),
        in_specs=[
            pl.BlockSpec((NB, 1), lambda i: (i, 0)),
            pl.BlockSpec((NB, POS_DIM), lambda i: (i, 0)),
            _full_spec((22, EMB_DIM)),
            _full_spec((1, EMB_DIM)),
            _full_spec((EMB_DIM, PROW)),
            _full_spec((EMB_DIM, PROW)),
        ],
        out_specs=(nspec, pspec, pspec),
        out_shape=(jax.ShapeDtypeStruct((N_NODES, TROW), f32),
                   jax.ShapeDtypeStruct((N_NODES, PROW), f32),
                   jax.ShapeDtypeStruct((N_NODES, PROW), f32)),
    )(z2d, pos, wf, bf, ks[0][0], ks[0][1])

    tables = [t0]
    t_cur = t0
    for li, (wa, wb, wtrig, op, ph, eb1, ew2, eb2, cw1, cb1, cw2, cb2,
             nw1a, nw1b0, nw1b1, nb1, nw2, nb2) in enumerate(ks):
        h_rows = _sc_gather(pa, pb, src, dst)

        e_grid = N_EDGES // EB
        msg = pl.pallas_call(
            _edge_tc,
            grid=(e_grid,),
            in_specs=[
                pl.BlockSpec((EB, PROW), lambda i: (i, 0)),
                _full_spec((16, 128)), _full_spec((PROW, 16)),
                _full_spec((1, 16)), _full_spec((1, 128)),
                _full_spec((128, M_DIM)), _full_spec((1, M_DIM)),
                _full_spec((M_DIM, M_DIM * 4)), _full_spec((1, M_DIM * 4)),
                _full_spec((M_DIM * 4, 8)), _full_spec((1, 8)),
            ],
            out_specs=pl.BlockSpec((EB, MROW), lambda i: (i, 0)),
            out_shape=jax.ShapeDtypeStruct((N_EDGES, MROW), f32),
        )(h_rows, wtrig, op, ph, eb1, ew2, eb2, cw1, cb1, cw2, cb2)

        acc0 = _sc_scatter(msg, dst, zeros_acc, 0)
        acc1 = _sc_scatter(msg, dst, zeros_acc, 1)

        nwa, nwb = (ks[li + 1][0], ks[li + 1][1]) if li + 1 < len(ks) \
            else (ks[li][0], ks[li][1])
        t_cur, pa, pb = pl.pallas_call(
            _node_tc,
            grid=(n_grid,),
            in_specs=[
                pl.BlockSpec((NB, TROW), lambda i: (i, 0)),
                pl.BlockSpec((NB, PASS_W), lambda i: (i, 0)),
                pl.BlockSpec((NB, PASS_W), lambda i: (i, 0)),
                _full_spec((EMB_DIM, EMB_DIM * 2)),
                _full_spec((PASS_W, EMB_DIM * 2)),
                _full_spec((M_DIM - PASS_W, EMB_DIM * 2)),
                _full_spec((1, EMB_DIM * 2)),
                _full_spec((EMB_DIM * 2, EMB_DIM)),
                _full_spec((1, EMB_DIM)),
                _full_spec((EMB_DIM, PROW)),
                _full_spec((EMB_DIM, PROW)),
            ],
            out_specs=(nspec, pspec, pspec),
            out_shape=(jax.ShapeDtypeStruct((N_NODES, TROW), f32),
                       jax.ShapeDtypeStruct((N_NODES, PROW), f32),
                       jax.ShapeDtypeStruct((N_NODES, PROW), f32)),
        )(t_cur, acc0, acc1, nw1a, nw1b0, nw1b1, nb1, nw2, nb2, nwa, nwb)
        tables.append(t_cur)

    out = pl.pallas_call(
        _final_tc,
        grid=(n_grid,),
        in_specs=[
            pl.BlockSpec((NB, TROW), lambda i: (i, 0)),
            pl.BlockSpec((NB, TROW), lambda i: (i, 0)),
            pl.BlockSpec((NB, TROW), lambda i: (i, 0)),
            pl.BlockSpec((NB, 1), lambda i: (i, 0)),
            _full_spec((EMB_DIM, MLP_DIM)), _full_spec((EMB_DIM, MLP_DIM)),
            _full_spec((EMB_DIM, MLP_DIM)), _full_spec((1, MLP_DIM)),
            _full_spec((MLP_DIM, MLP_DIM)), _full_spec((1, MLP_DIM)),
            _full_spec((MLP_DIM, MLP_DIM)), _full_spec((1, MLP_DIM)),
            _full_spec((MLP_DIM, N_OUT)), _full_spec((1, N_OUT)),
        ],
        out_specs=pl.BlockSpec((N_GRAPHS, N_OUT), lambda i: (0, 0)),
        out_shape=jax.ShapeDtypeStruct((N_GRAPHS, N_OUT), f32),
        scratch_shapes=[
            pltpu.VMEM((N_GRAPHS, MLP_DIM), f32),
            pltpu.VMEM((1, N_GRAPHS), f32),
        ],
    )(tables[0], tables[1], tables[2], bid2d,
      w1a, w1b, w1c, fb1, fw2, fb2, fw3, fb3, fw4, fb4)
    return out
